# baseline (speedup 1.0000x reference)
import functools

import jax
import jax.numpy as jnp
from jax import lax
from jax.experimental import pallas as pl
from jax.experimental.pallas import tpu as pltpu
from jax.experimental.pallas import tpu_sc as plsc

N = 10000
E = 320000
D = 128
H = 128
EA = 64
TL = 128
G = 16

NC = 2
NS = 16
NW = NC * NS

def _ln(x, g, b, eps=1e-5):
    m = jnp.mean(x, axis=-1, keepdims=True)
    v = jnp.mean((x - m) ** 2, axis=-1, keepdims=True)
    return (x - m) / jnp.sqrt(v + eps) * g + b


_BE = 1280


def _edge_pre_body(ta_ref, ax_ref, d_ref, b_ref, tw_ref, tb_ref, aw_ref, ab_ref,
                   eg_ref, eb_ref, w1a_ref, b1a_ref, w1b_ref, b1b_ref,
                   ww1_ref, wb1_ref, wg1_ref, wbt1_ref,
                   ww2_ref, wb2_ref, wg2_ref, wbt2_ref,
                   ww3_ref, wb3_ref,
                   e2a_ref, e2b_ref, ew_ref, cnt_ref, bnd_ref):
    @pl.when(pl.program_id(0) == 0)
    def _init():
        cnt_ref[...] = jnp.zeros_like(cnt_ref)
        b = b_ref[...]
        iota = jax.lax.broadcasted_iota(jnp.int32, (1, 128), 1)
        acc = jnp.zeros((1, 128), jnp.int32)
        run = jnp.zeros((), jnp.int32)
        for g in range(G):
            acc = acc + jnp.where(iota == g, run, 0)
            run = run + jnp.sum((b == g).astype(jnp.int32))
        acc = acc + jnp.where(iota == G, run, 0)
        bnd_ref[...] = acc

    d = d_ref[...]
    dlo = (d & 127) == jax.lax.broadcasted_iota(jnp.int32, (1, 128), 1)
    dhi = (d >> 7) == jax.lax.broadcasted_iota(jnp.int32, (1, N_PAD // 128), 1)
    cnt_ref[...] += jax.lax.dot_general(
        dhi.astype(jnp.float32), dlo.astype(jnp.float32),
        (((0,), (0,)), ((), ())), preferred_element_type=jnp.float32)

    ta = ta_ref[...]
    ax = ax_ref[...]
    tf = jnp.dot(ta, tw_ref[...], preferred_element_type=jnp.float32) + tb_ref[...]
    af = jnp.dot(ax, aw_ref[...], preferred_element_type=jnp.float32) + ab_ref[...]
    ef = jnp.concatenate([tf, af], axis=-1)
    ef = _ln(ef, eg_ref[...], eb_ref[...])
    e2a_ref[...] = jnp.dot(ef, w1a_ref[...], preferred_element_type=jnp.float32) + b1a_ref[...]
    e2b_ref[...] = jnp.dot(ef, w1b_ref[...], preferred_element_type=jnp.float32) + b1b_ref[...]
    w1 = jax.nn.relu(_ln(jnp.dot(ax, ww1_ref[...], preferred_element_type=jnp.float32) + wb1_ref[...],
                         wg1_ref[...], wbt1_ref[...]))
    w2 = jax.nn.relu(_ln(jnp.dot(w1, ww2_ref[...], preferred_element_type=jnp.float32) + wb2_ref[...],
                         wg2_ref[...], wbt2_ref[...]))
    ew_ref[...] = jax.nn.sigmoid(
        jnp.dot(w2, ww3_ref[...], preferred_element_type=jnp.float32) + wb3_ref[...])


def _full(shape):
    return pl.BlockSpec(shape, lambda i: (0, 0))


def _edge_pre(time_attr, aux_info, dst_r, batch_pad, p):
    grid = (E // _BE,)
    row = lambda c: pl.BlockSpec((_BE, c), lambda i: (i, 0))
    return pl.pallas_call(
        _edge_pre_body,
        grid=grid,
        in_specs=[
            row(64), row(2), row(1),
            pl.BlockSpec((N_PAD // 128, 128), lambda i: (0, 0)),
            _full((64, 32)), _full((1, 32)), _full((2, 32)), _full((1, 32)),
            _full((1, 64)), _full((1, 64)),
            _full((64, 128)), _full((1, 128)), _full((64, 128)), _full((1, 128)),
            _full((2, 128)), _full((1, 128)), _full((1, 128)), _full((1, 128)),
            _full((128, 128)), _full((1, 128)), _full((1, 128)), _full((1, 128)),
            _full((128, 1)), _full((1, 1)),
        ],
        out_specs=[row(128), row(128), row(1),
                   pl.BlockSpec((N_PAD // 128, 128), lambda i: (0, 0)),
                   pl.BlockSpec((1, 128), lambda i: (0, 0))],
        out_shape=[
            jax.ShapeDtypeStruct((E, 128), jnp.float32),
            jax.ShapeDtypeStruct((E, 128), jnp.float32),
            jax.ShapeDtypeStruct((E, 1), jnp.float32),
            jax.ShapeDtypeStruct((N_PAD // 128, 128), jnp.float32),
            jax.ShapeDtypeStruct((1, 128), jnp.int32),
        ],
    )(time_attr, aux_info, dst_r, batch_pad,
      p['time_proj_w'], p['time_proj_b'].reshape(1, -1),
      p['aux_proj_w'], p['aux_proj_b'].reshape(1, -1),
      p['edge_norm_g'].reshape(1, -1), p['edge_norm_b'].reshape(1, -1),
      p['conv1']['lin_edge_w'], p['conv1']['lin_edge_b'].reshape(1, -1),
      p['conv2']['lin_edge_w'], p['conv2']['lin_edge_b'].reshape(1, -1),
      p['wn_w1'], p['wn_b1'].reshape(1, -1), p['wn_ln1_g'].reshape(1, -1), p['wn_ln1_b'].reshape(1, -1),
      p['wn_w2'], p['wn_b2'].reshape(1, -1), p['wn_ln2_g'].reshape(1, -1), p['wn_ln2_b'].reshape(1, -1),
      p['wn_w3'], p['wn_b3'].reshape(1, -1))


_BN = 1000


def _node_body(p_ref, sc_ref, x_ref, sa_ref, sb_ref, cn_ref,
               w1_ref, b1_ref, lg_ref, lb_ref, w2_ref, b2_ref, out_ref):
    x = x_ref[...]
    s = sa_ref[...] + sb_ref[...]
    cnt = cn_ref[...]
    mean = s / jnp.maximum(cnt, 1.0)
    mean = jnp.clip(mean, 1e-7, 100.0)
    agg = jnp.exp(jnp.log(mean) / p_ref[0])
    nrm = jnp.sqrt(jnp.sum(agg * agg, axis=-1, keepdims=True))
    mn = agg / jnp.maximum(nrm, 1e-12)
    xn = jnp.sqrt(jnp.sum(x * x, axis=-1, keepdims=True))
    out = x + mn * xn * sc_ref[0]
    t = jnp.dot(out, w1_ref[...], preferred_element_type=jnp.float32) + b1_ref[...]
    t = jax.nn.relu(_ln(t, lg_ref[...], lb_ref[...]))
    out_ref[...] = jnp.dot(t, w2_ref[...], preferred_element_type=jnp.float32) + b2_ref[...]


def _node_update(x, s_a, s_b, cnt, conv):
    row = lambda c: pl.BlockSpec((_BN, c), lambda i: (i, 0))
    return pl.pallas_call(
        _node_body,
        grid=(N // _BN,),
        in_specs=[
            pl.BlockSpec(memory_space=pltpu.SMEM),
            pl.BlockSpec(memory_space=pltpu.SMEM),
            row(128), row(128), row(128), row(1),
            _full((128, 256)), _full((1, 256)), _full((1, 256)), _full((1, 256)),
            _full((256, 128)), _full((1, 128)),
        ],
        out_specs=row(128),
        out_shape=jax.ShapeDtypeStruct((N, 128), jnp.float32),
    )(conv['p'].reshape(1), conv['scale'].reshape(1),
      x, s_a, s_b, cnt,
      conv['mlp_w1'], conv['mlp_b1'].reshape(1, -1),
      conv['mlp_ln_g'].reshape(1, -1), conv['mlp_ln_b'].reshape(1, -1),
      conv['mlp_w2'], conv['mlp_b2'].reshape(1, -1))


_BF = 2560


def _fmax_body(hs_ref, hd_ref, ew_ref, src_ref, bnd_ref, out_ref):
    @pl.when(pl.program_id(0) == 0)
    def _init():
        out_ref[...] = jnp.full_like(out_ref, -jnp.inf)

    w = jnp.concatenate([hs_ref[...], hd_ref[...]], axis=-1) * ew_ref[...]
    s = src_ref[...]
    bnd = bnd_ref[...]
    gid = jnp.zeros_like(s)
    for g in range(1, G):
        gid = gid + (s >= bnd[0, g]).astype(jnp.int32)
    rows = []
    for g in range(G):
        cand = jnp.where(gid == g, w, -jnp.inf)
        rows.append(jnp.max(cand, axis=0, keepdims=True))
    out_ref[...] = jnp.maximum(out_ref[...], jnp.concatenate(rows, axis=0))


def _fusion_max(hs, hd, ew, src, bnd):
    row = lambda c: pl.BlockSpec((_BF, c), lambda i: (i, 0))
    return pl.pallas_call(
        _fmax_body,
        grid=(E // _BF,),
        in_specs=[row(128), row(128), row(1), row(1),
                  pl.BlockSpec((1, 128), lambda i: (0, 0))],
        out_specs=pl.BlockSpec((G, 256), lambda i: (0, 0)),
        out_shape=jax.ShapeDtypeStruct((G, 256), jnp.float32),
    )(hs, hd, ew, src, bnd)


def _fhead_body(f_ref, g_ref, b_ref, w_ref, wb_ref, out_ref):
    t = _ln(f_ref[...], g_ref[...], b_ref[...])
    out_ref[...] = jax.nn.sigmoid(
        jnp.dot(t, w_ref[...], preferred_element_type=jnp.float32) + wb_ref[...])


def _fusion_head(fmax, p):
    return pl.pallas_call(
        _fhead_body,
        grid=(1,),
        in_specs=[_full((G, 256)),
                  _full((1, 256)), _full((1, 256)),
                  _full((256, TL)), _full((1, TL))],
        out_specs=_full((G, TL)),
        out_shape=jax.ShapeDtypeStruct((G, TL), jnp.float32),
    )(fmax, p['norm_g'].reshape(1, -1), p['norm_b'].reshape(1, -1),
      p['fuse_w'], p['fuse_b'].reshape(1, -1))


def _contr_a_body(h_ref, w_ref, b_ref, y_ref, st_ref):
    @pl.when(pl.program_id(0) == 0)
    def _init():
        st_ref[...] = jnp.zeros_like(st_ref)

    y = jnp.dot(h_ref[...], w_ref[...], preferred_element_type=jnp.float32) + b_ref[...]
    y_ref[...] = y
    st_ref[0:1, :] += jnp.sum(y, axis=0, keepdims=True)
    st_ref[1:2, :] += jnp.sum(y * y, axis=0, keepdims=True)


def _contr_b_body(y_ref, st_ref, g_ref, b_ref, w_ref, wb_ref, out_ref):
    mean = st_ref[0:1, :] / N
    var = st_ref[1:2, :] / N - mean * mean
    xh = (y_ref[...] - mean) / jnp.sqrt(var + 1e-5) * g_ref[...] + b_ref[...]
    xh = jax.nn.relu(xh)
    out_ref[...] = jnp.dot(xh, w_ref[...], preferred_element_type=jnp.float32) + wb_ref[...]


def _contrastive(h, p):
    row = lambda c: pl.BlockSpec((_BN, c), lambda i: (i, 0))
    y, st = pl.pallas_call(
        _contr_a_body,
        grid=(N // _BN,),
        in_specs=[row(128), _full((128, 128)), _full((1, 128))],
        out_specs=[row(128), pl.BlockSpec((8, 128), lambda i: (0, 0))],
        out_shape=[jax.ShapeDtypeStruct((N, 128), jnp.float32),
                   jax.ShapeDtypeStruct((8, 128), jnp.float32)],
    )(h, p['c_w1'], p['c_b1'].reshape(1, -1))
    return pl.pallas_call(
        _contr_b_body,
        grid=(N // _BN,),
        in_specs=[row(128), _full((8, 128)), _full((1, 128)), _full((1, 128)),
                  _full((128, 128)), _full((1, 128))],
        out_specs=row(128),
        out_shape=jax.ShapeDtypeStruct((N, 128), jnp.float32),
    )(y, st, p['c_bn_g'].reshape(1, -1), p['c_bn_b'].reshape(1, -1),
      p['c_w2'], p['c_b2'].reshape(1, -1))


def _recon_body(h_ref, w1_ref, b1_ref, w2_ref, b2_ref,
                w3_ref, b3_ref, w4_ref, b4_ref, out_ref):
    t = jax.nn.relu(jnp.dot(h_ref[...], w1_ref[...], preferred_element_type=jnp.float32) + b1_ref[...])
    enc = jnp.dot(t, w2_ref[...], preferred_element_type=jnp.float32) + b2_ref[...]
    t2 = jax.nn.relu(jnp.dot(enc, w3_ref[...], preferred_element_type=jnp.float32) + b3_ref[...])
    out_ref[...] = jnp.dot(t2, w4_ref[...], preferred_element_type=jnp.float32) + b4_ref[...]


def _recon(h, p):
    row = lambda c: pl.BlockSpec((_BN, c), lambda i: (i, 0))
    return pl.pallas_call(
        _recon_body,
        grid=(N // _BN,),
        in_specs=[row(128),
                  _full((128, 128)), _full((1, 128)),
                  _full((128, 64)), _full((1, 64)),
                  _full((64, 128)), _full((1, 128)),
                  _full((128, 128)), _full((1, 128))],
        out_specs=row(128),
        out_shape=jax.ShapeDtypeStruct((N, 128), jnp.float32),
    )(h, p['r_e_w1'], p['r_e_b1'].reshape(1, -1),
      p['r_e_w2'], p['r_e_b2'].reshape(1, -1),
      p['r_d_w1'], p['r_d_b1'].reshape(1, -1),
      p['r_d_w2'], p['r_d_b2'].reshape(1, -1))


_MESH = plsc.VectorSubcoreMesh(core_axis_name="c", subcore_axis_name="s",
                               num_cores=NC, num_subcores=NS)
_C = 80
_RPW = E // NW
_NST = _RPW // _C
N_PAD = 10240
_RPT = N_PAD // NS


_CV = 40
_NSV = _RPW // _CV


def _sc_conv(table, src, dst, e2, zeros):
    @functools.partial(
        pl.kernel, mesh=_MESH,
        out_type=jax.ShapeDtypeStruct((NC, N_PAD, 128), jnp.float32),
        scratch_types=[pltpu.VMEM((_CV,), jnp.int32),
                       pltpu.VMEM((_CV,), jnp.int32),
                       pltpu.VMEM((_CV,), jnp.int32),
                       pltpu.VMEM((_CV,), jnp.int32),
                       pltpu.VMEM((_CV, 128), jnp.float32),
                       pltpu.VMEM((_CV, 128), jnp.float32),
                       pltpu.VMEM((_CV, 128), jnp.float32),
                       pltpu.VMEM((_CV, 128), jnp.float32),
                       pltpu.VMEM_SHARED((N_PAD, 128), jnp.float32),
                       pltpu.SemaphoreType.DMA,
                       pltpu.SemaphoreType.DMA,
                       pltpu.SemaphoreType.DMA],
    )
    def k(tab_hbm, src_hbm, dst_hbm, e2_hbm, zero_hbm, sum_hbm,
          s0, s1, d0, d1, x0, x1, e0, e1, acc, gsem, esem, ssem):
        cid = lax.axis_index("c")
        sid = lax.axis_index("s")
        wid = sid * NC + cid
        pltpu.sync_copy(zero_hbm, acc.at[pl.ds(sid * _RPT, _RPT)])
        plsc.subcore_barrier()
        bufs = ((s0, d0, x0, e0), (s1, d1, x1, e1))

        def start(t, sb, xb, eb):
            base = wid * _RPW + t * _CV
            pltpu.sync_copy(src_hbm.at[pl.ds(base, _CV)], sb)
            pltpu.async_copy(e2_hbm.at[pl.ds(base, _CV)], eb, esem)
            pltpu.async_copy(tab_hbm.at[sb], xb, gsem)

        for b in range(2):
            start(b, bufs[b][0], bufs[b][2], bufs[b][3])

        def process(t, sb, db, xb, eb, drain, prefetch):
            base = wid * _RPW + t * _CV
            pltpu.make_async_copy(tab_hbm.at[sb], xb, gsem).wait()
            pltpu.make_async_copy(e2_hbm.at[pl.ds(base, _CV)], eb, esem).wait()
            if drain:
                @pl.when(t >= 2)
                def _dr():
                    pltpu.make_async_copy(xb, acc.at[db], ssem).wait()
            pltpu.sync_copy(dst_hbm.at[pl.ds(base, _CV)], db)

            def row(r, carry):
                for c in range(8):
                    sl = pl.ds(c * 16, 16)
                    v = jnp.maximum(xb[r, sl] + eb[r, sl], 0.0) + 1e-7
                    xb[r, sl] = jnp.minimum(v, 100.0)
                return carry

            lax.fori_loop(0, _CV, row, 0)
            pltpu.async_copy(xb, acc.at[db], ssem, add=True)
            if prefetch:
                @pl.when(t + 2 < _NSV)
                def _pf():
                    start(t + 2, sb, xb, eb)

        def pair(kk, carry):
            for b in range(2):
                t = 2 * kk + b
                process(t, *bufs[b], drain=True, prefetch=True)
            return carry

        lax.fori_loop(0, _NSV // 2 - 1, pair, 0)
        for t in (_NSV - 2, _NSV - 1):
            process(t, *bufs[t % 2], drain=True, prefetch=False)
        for t in (_NSV - 2, _NSV - 1):
            sb, db, xb, eb = bufs[t % 2]
            pltpu.make_async_copy(xb, acc.at[db], ssem).wait()
        plsc.subcore_barrier()
        pltpu.sync_copy(acc.at[pl.ds(sid * _RPT, _RPT)],
                        sum_hbm.at[cid, pl.ds(sid * _RPT, _RPT)])

    return k(table, src, dst, e2, zeros)


_CF = 80
_NSF = _RPW // _CF


def _sc_gather_fusion(h, src, dst):
    @functools.partial(
        pl.kernel, mesh=_MESH,
        out_type=[jax.ShapeDtypeStruct((E, 128), jnp.float32),
                  jax.ShapeDtypeStruct((E, 128), jnp.float32)],
        scratch_types=[pltpu.VMEM((_CF,), jnp.int32),
                       pltpu.VMEM((_CF,), jnp.int32),
                       pltpu.VMEM((_CF,), jnp.int32),
                       pltpu.VMEM((_CF,), jnp.int32),
                       pltpu.VMEM((_CF, 128), jnp.float32),
                       pltpu.VMEM((_CF, 128), jnp.float32),
                       pltpu.VMEM((_CF, 128), jnp.float32),
                       pltpu.VMEM((_CF, 128), jnp.float32),
                       pltpu.SemaphoreType.DMA],
    )
    def k(h_hbm, src_hbm, dst_hbm, hs_hbm, hd_hbm,
          s0, s1, d0, d1, hs0, hs1, hd0, hd1, sem):
        wid = lax.axis_index("s") * NC + lax.axis_index("c")
        bufs = ((s0, d0, hs0, hd0), (s1, d1, hs1, hd1))

        def start(t, sb, db, hsb, hdb):
            base = wid * _RPW + t * _CF
            pltpu.sync_copy(src_hbm.at[pl.ds(base, _CF)], sb)
            pltpu.sync_copy(dst_hbm.at[pl.ds(base, _CF)], db)
            pltpu.async_copy(h_hbm.at[sb], hsb, sem)
            pltpu.async_copy(h_hbm.at[db], hdb, sem)

        for b in range(2):
            start(b, *bufs[b])

        def finish(t, sb, db, hsb, hdb):
            pltpu.make_async_copy(h_hbm.at[sb], hsb, sem).wait()
            pltpu.make_async_copy(h_hbm.at[db], hdb, sem).wait()
            base = wid * _RPW + t * _CF
            pltpu.sync_copy(hsb, hs_hbm.at[pl.ds(base, _CF)])
            pltpu.sync_copy(hdb, hd_hbm.at[pl.ds(base, _CF)])

        def pair(kk, carry):
            for b in range(2):
                t = 2 * kk + b
                finish(t, *bufs[b])

                @pl.when(t + 2 < _NSF)
                def _pf():
                    start(t + 2, *bufs[b])
            return carry

        lax.fori_loop(0, (_NSF - 1) // 2, pair, 0)
        tl = _NSF - 1
        finish(tl, *bufs[tl % 2])

    return k(h, src, dst)


def kernel(x, edge_index, time_attr, aux_info, pos, batch, params):
    p = params
    src, dst = edge_index[0], edge_index[1]
    zeros = jnp.zeros((_RPT, 128), jnp.float32)
    batch_pad = jnp.pad(batch, (0, N_PAD - N), constant_values=G).reshape(N_PAD // 128, 128)
    e2_1, e2_2, ew, cnt128, bnd = _edge_pre(
        time_attr, aux_info, dst.reshape(E, 1), batch_pad, p)
    cnt = cnt128.reshape(N_PAD, 1)[:N]

    sums1 = _sc_conv(x, src, dst, e2_1, zeros)
    h1 = _node_update(x, sums1[0, :N], sums1[1, :N], cnt, p['conv1'])

    sums2 = _sc_conv(h1, src, dst, e2_2, zeros)
    h = _node_update(h1, sums2[0, :N], sums2[1, :N], cnt, p['conv2'])

    hs, hd = _sc_gather_fusion(h, src, dst)
    fmax = _fusion_max(hs, hd, ew, src.reshape(E, 1), bnd)
    fusion = _fusion_head(fmax, p)

    contrastive = _contrastive(h, p)
    recon = _recon(h, p)
    return (h, fusion, contrastive, recon)


# speedup vs baseline: 1.0382x; 1.0382x over previous
import functools

import jax
import jax.numpy as jnp
from jax import lax
from jax.experimental import pallas as pl
from jax.experimental.pallas import tpu as pltpu
from jax.experimental.pallas import tpu_sc as plsc

N = 10000
E = 320000
D = 128
H = 128
EA = 64
TL = 128
G = 16

NC = 2
NS = 16
NW = NC * NS

def _ln(x, g, b, eps=1e-5):
    m = jnp.mean(x, axis=-1, keepdims=True)
    v = jnp.mean((x - m) ** 2, axis=-1, keepdims=True)
    return (x - m) / jnp.sqrt(v + eps) * g + b


_BE = 1280


def _edge_pre_body(ta_ref, ax_ref, tw_ref, tb_ref, aw_ref, ab_ref,
                   eg_ref, eb_ref, w1a_ref, b1a_ref, w1b_ref, b1b_ref,
                   ww1_ref, wb1_ref, wg1_ref, wbt1_ref,
                   ww2_ref, wb2_ref, wg2_ref, wbt2_ref,
                   ww3_ref, wb3_ref,
                   e2a_ref, e2b_ref, ew_ref):
    ta = ta_ref[...]
    ax = ax_ref[...]
    tf = jnp.dot(ta, tw_ref[...], preferred_element_type=jnp.float32) + tb_ref[...]
    af = jnp.dot(ax, aw_ref[...], preferred_element_type=jnp.float32) + ab_ref[...]
    ef = jnp.concatenate([tf, af], axis=-1)
    ef = _ln(ef, eg_ref[...], eb_ref[...])
    e2a_ref[...] = jnp.dot(ef, w1a_ref[...], preferred_element_type=jnp.float32) + b1a_ref[...]
    e2b_ref[...] = jnp.dot(ef, w1b_ref[...], preferred_element_type=jnp.float32) + b1b_ref[...]
    w1 = jax.nn.relu(_ln(jnp.dot(ax, ww1_ref[...], preferred_element_type=jnp.float32) + wb1_ref[...],
                         wg1_ref[...], wbt1_ref[...]))
    w2 = jax.nn.relu(_ln(jnp.dot(w1, ww2_ref[...], preferred_element_type=jnp.float32) + wb2_ref[...],
                         wg2_ref[...], wbt2_ref[...]))
    ew_ref[...] = jax.nn.sigmoid(
        jnp.dot(w2, ww3_ref[...], preferred_element_type=jnp.float32) + wb3_ref[...])


def _full(shape):
    return pl.BlockSpec(shape, lambda i: (0, 0))


def _edge_pre(time_attr, aux_info, p):
    grid = (E // _BE,)
    row = lambda c: pl.BlockSpec((_BE, c), lambda i: (i, 0))
    return pl.pallas_call(
        _edge_pre_body,
        grid=grid,
        in_specs=[
            row(64), row(2),
            _full((64, 32)), _full((1, 32)), _full((2, 32)), _full((1, 32)),
            _full((1, 64)), _full((1, 64)),
            _full((64, 128)), _full((1, 128)), _full((64, 128)), _full((1, 128)),
            _full((2, 128)), _full((1, 128)), _full((1, 128)), _full((1, 128)),
            _full((128, 128)), _full((1, 128)), _full((1, 128)), _full((1, 128)),
            _full((128, 1)), _full((1, 1)),
        ],
        out_specs=[row(128), row(128), row(1)],
        out_shape=[
            jax.ShapeDtypeStruct((E, 128), jnp.float32),
            jax.ShapeDtypeStruct((E, 128), jnp.float32),
            jax.ShapeDtypeStruct((E, 1), jnp.float32),
        ],
    )(time_attr, aux_info,
      p['time_proj_w'], p['time_proj_b'].reshape(1, -1),
      p['aux_proj_w'], p['aux_proj_b'].reshape(1, -1),
      p['edge_norm_g'].reshape(1, -1), p['edge_norm_b'].reshape(1, -1),
      p['conv1']['lin_edge_w'], p['conv1']['lin_edge_b'].reshape(1, -1),
      p['conv2']['lin_edge_w'], p['conv2']['lin_edge_b'].reshape(1, -1),
      p['wn_w1'], p['wn_b1'].reshape(1, -1), p['wn_ln1_g'].reshape(1, -1), p['wn_ln1_b'].reshape(1, -1),
      p['wn_w2'], p['wn_b2'].reshape(1, -1), p['wn_ln2_g'].reshape(1, -1), p['wn_ln2_b'].reshape(1, -1),
      p['wn_w3'], p['wn_b3'].reshape(1, -1))


_BN = 1000


def _node_body(p_ref, sc_ref, x_ref, sa_ref, sb_ref, cn_ref,
               w1_ref, b1_ref, lg_ref, lb_ref, w2_ref, b2_ref, out_ref):
    x = x_ref[...]
    s = sa_ref[...] + sb_ref[...]
    cnt = cn_ref[...]
    mean = s / jnp.maximum(cnt, 1.0)
    mean = jnp.clip(mean, 1e-7, 100.0)
    agg = jnp.exp(jnp.log(mean) / p_ref[0])
    nrm = jnp.sqrt(jnp.sum(agg * agg, axis=-1, keepdims=True))
    mn = agg / jnp.maximum(nrm, 1e-12)
    xn = jnp.sqrt(jnp.sum(x * x, axis=-1, keepdims=True))
    out = x + mn * xn * sc_ref[0]
    t = jnp.dot(out, w1_ref[...], preferred_element_type=jnp.float32) + b1_ref[...]
    t = jax.nn.relu(_ln(t, lg_ref[...], lb_ref[...]))
    out_ref[...] = jnp.dot(t, w2_ref[...], preferred_element_type=jnp.float32) + b2_ref[...]


def _node_update(x, s_a, s_b, cnt, conv):
    row = lambda c: pl.BlockSpec((_BN, c), lambda i: (i, 0))
    return pl.pallas_call(
        _node_body,
        grid=(N // _BN,),
        in_specs=[
            pl.BlockSpec(memory_space=pltpu.SMEM),
            pl.BlockSpec(memory_space=pltpu.SMEM),
            row(128), row(128), row(128), row(1),
            _full((128, 256)), _full((1, 256)), _full((1, 256)), _full((1, 256)),
            _full((256, 128)), _full((1, 128)),
        ],
        out_specs=row(128),
        out_shape=jax.ShapeDtypeStruct((N, 128), jnp.float32),
    )(conv['p'].reshape(1), conv['scale'].reshape(1),
      x, s_a, s_b, cnt,
      conv['mlp_w1'], conv['mlp_b1'].reshape(1, -1),
      conv['mlp_ln_g'].reshape(1, -1), conv['mlp_ln_b'].reshape(1, -1),
      conv['mlp_w2'], conv['mlp_b2'].reshape(1, -1))


def _counts_body(d_ref, b_ref, cnt_ref, bnd_ref):
    @pl.when(pl.program_id(0) == 0)
    def _init():
        cnt_ref[...] = jnp.zeros_like(cnt_ref)
        b = b_ref[...]
        iota = jax.lax.broadcasted_iota(jnp.int32, (1, 128), 1)
        acc = jnp.zeros((1, 128), jnp.int32)
        run = jnp.zeros((), jnp.int32)
        for g in range(G):
            acc = acc + jnp.where(iota == g, run, 0)
            run = run + jnp.sum((b == g).astype(jnp.int32))
        acc = acc + jnp.where(iota == G, run, 0)
        bnd_ref[...] = acc

    d = d_ref[...]
    dlo = (d & 127) == jax.lax.broadcasted_iota(jnp.int32, (1, 128), 1)
    dhi = (d >> 7) == jax.lax.broadcasted_iota(jnp.int32, (1, N_PAD // 128), 1)
    cnt_ref[...] += jax.lax.dot_general(
        dhi.astype(jnp.float32), dlo.astype(jnp.float32),
        (((0,), (0,)), ((), ())), preferred_element_type=jnp.float32)


def _counts(dst, batch_pad):
    return pl.pallas_call(
        _counts_body,
        grid=(E // _BE,),
        in_specs=[pl.BlockSpec((_BE, 1), lambda i: (i, 0)),
                  pl.BlockSpec((N_PAD // 128, 128), lambda i: (0, 0))],
        out_specs=[pl.BlockSpec((N_PAD // 128, 128), lambda i: (0, 0)),
                   pl.BlockSpec((1, 128), lambda i: (0, 0))],
        out_shape=[jax.ShapeDtypeStruct((N_PAD // 128, 128), jnp.float32),
                   jax.ShapeDtypeStruct((1, 128), jnp.int32)],
    )(dst, batch_pad)


_BF = 1280


def _fmax_body(hs_ref, hd_ref, ew_ref, src_ref, bnd_ref, out_ref):
    @pl.when(pl.program_id(0) == 0)
    def _init():
        out_ref[...] = jnp.full_like(out_ref, -jnp.inf)

    w = jnp.concatenate([hs_ref[...], hd_ref[...]], axis=-1) * ew_ref[...]
    s = src_ref[...]
    bnd = bnd_ref[...]
    gid = jnp.zeros_like(s)
    for g in range(1, G):
        gid = gid + (s >= bnd[0, g]).astype(jnp.int32)
    rows = []
    for g in range(G):
        cand = jnp.where(gid == g, w, -jnp.inf)
        rows.append(jnp.max(cand, axis=0, keepdims=True))
    out_ref[...] = jnp.maximum(out_ref[...], jnp.concatenate(rows, axis=0))


def _fusion_max(hs, hd, ew, src, bnd):
    row = lambda c: pl.BlockSpec((_BF, c), lambda i: (i, 0))
    return pl.pallas_call(
        _fmax_body,
        grid=(E // _BF,),
        in_specs=[row(128), row(128), row(1), row(1),
                  pl.BlockSpec((1, 128), lambda i: (0, 0))],
        out_specs=pl.BlockSpec((G, 256), lambda i: (0, 0)),
        out_shape=jax.ShapeDtypeStruct((G, 256), jnp.float32),
    )(hs, hd, ew, src, bnd)


def _fhead_body(f_ref, g_ref, b_ref, w_ref, wb_ref, out_ref):
    t = _ln(f_ref[...], g_ref[...], b_ref[...])
    out_ref[...] = jax.nn.sigmoid(
        jnp.dot(t, w_ref[...], preferred_element_type=jnp.float32) + wb_ref[...])


def _fusion_head(fmax, p):
    return pl.pallas_call(
        _fhead_body,
        grid=(1,),
        in_specs=[_full((G, 256)),
                  _full((1, 256)), _full((1, 256)),
                  _full((256, TL)), _full((1, TL))],
        out_specs=_full((G, TL)),
        out_shape=jax.ShapeDtypeStruct((G, TL), jnp.float32),
    )(fmax, p['norm_g'].reshape(1, -1), p['norm_b'].reshape(1, -1),
      p['fuse_w'], p['fuse_b'].reshape(1, -1))


def _contr_a_body(h_ref, w_ref, b_ref, y_ref, st_ref):
    @pl.when(pl.program_id(0) == 0)
    def _init():
        st_ref[...] = jnp.zeros_like(st_ref)

    y = jnp.dot(h_ref[...], w_ref[...], preferred_element_type=jnp.float32) + b_ref[...]
    y_ref[...] = y
    st_ref[0:1, :] += jnp.sum(y, axis=0, keepdims=True)
    st_ref[1:2, :] += jnp.sum(y * y, axis=0, keepdims=True)


def _contr_b_body(y_ref, st_ref, g_ref, b_ref, w_ref, wb_ref, out_ref):
    mean = st_ref[0:1, :] / N
    var = st_ref[1:2, :] / N - mean * mean
    xh = (y_ref[...] - mean) / jnp.sqrt(var + 1e-5) * g_ref[...] + b_ref[...]
    xh = jax.nn.relu(xh)
    out_ref[...] = jnp.dot(xh, w_ref[...], preferred_element_type=jnp.float32) + wb_ref[...]


def _contrastive(h, p):
    row = lambda c: pl.BlockSpec((_BN, c), lambda i: (i, 0))
    y, st = pl.pallas_call(
        _contr_a_body,
        grid=(N // _BN,),
        in_specs=[row(128), _full((128, 128)), _full((1, 128))],
        out_specs=[row(128), pl.BlockSpec((8, 128), lambda i: (0, 0))],
        out_shape=[jax.ShapeDtypeStruct((N, 128), jnp.float32),
                   jax.ShapeDtypeStruct((8, 128), jnp.float32)],
    )(h, p['c_w1'], p['c_b1'].reshape(1, -1))
    return pl.pallas_call(
        _contr_b_body,
        grid=(N // _BN,),
        in_specs=[row(128), _full((8, 128)), _full((1, 128)), _full((1, 128)),
                  _full((128, 128)), _full((1, 128))],
        out_specs=row(128),
        out_shape=jax.ShapeDtypeStruct((N, 128), jnp.float32),
    )(y, st, p['c_bn_g'].reshape(1, -1), p['c_bn_b'].reshape(1, -1),
      p['c_w2'], p['c_b2'].reshape(1, -1))


def _recon_body(h_ref, w1_ref, b1_ref, w2_ref, b2_ref,
                w3_ref, b3_ref, w4_ref, b4_ref, out_ref):
    t = jax.nn.relu(jnp.dot(h_ref[...], w1_ref[...], preferred_element_type=jnp.float32) + b1_ref[...])
    enc = jnp.dot(t, w2_ref[...], preferred_element_type=jnp.float32) + b2_ref[...]
    t2 = jax.nn.relu(jnp.dot(enc, w3_ref[...], preferred_element_type=jnp.float32) + b3_ref[...])
    out_ref[...] = jnp.dot(t2, w4_ref[...], preferred_element_type=jnp.float32) + b4_ref[...]


def _recon(h, p):
    row = lambda c: pl.BlockSpec((_BN, c), lambda i: (i, 0))
    return pl.pallas_call(
        _recon_body,
        grid=(N // _BN,),
        in_specs=[row(128),
                  _full((128, 128)), _full((1, 128)),
                  _full((128, 64)), _full((1, 64)),
                  _full((64, 128)), _full((1, 128)),
                  _full((128, 128)), _full((1, 128))],
        out_specs=row(128),
        out_shape=jax.ShapeDtypeStruct((N, 128), jnp.float32),
    )(h, p['r_e_w1'], p['r_e_b1'].reshape(1, -1),
      p['r_e_w2'], p['r_e_b2'].reshape(1, -1),
      p['r_d_w1'], p['r_d_b1'].reshape(1, -1),
      p['r_d_w2'], p['r_d_b2'].reshape(1, -1))


_MESH = plsc.VectorSubcoreMesh(core_axis_name="c", subcore_axis_name="s",
                               num_cores=NC, num_subcores=NS)
_C = 80
_RPW = E // NW
_NST = _RPW // _C
N_PAD = 10240
_RPT = N_PAD // NS


_CV = 80
_NSV = _RPW // _CV


def _sc_conv(table, src, dst, e2, zeros):
    @functools.partial(
        pl.kernel, mesh=_MESH,
        out_type=jax.ShapeDtypeStruct((NC, N_PAD, 128), jnp.float32),
        scratch_types=[pltpu.VMEM((_CV,), jnp.int32),
                       pltpu.VMEM((_CV,), jnp.int32),
                       pltpu.VMEM((_CV,), jnp.int32),
                       pltpu.VMEM((_CV,), jnp.int32),
                       pltpu.VMEM((_CV, 128), jnp.float32),
                       pltpu.VMEM((_CV, 128), jnp.float32),
                       pltpu.VMEM((_CV, 128), jnp.float32),
                       pltpu.VMEM_SHARED((N_PAD, 128), jnp.float32),
                       pltpu.SemaphoreType.DMA,
                       pltpu.SemaphoreType.DMA,
                       pltpu.SemaphoreType.DMA],
    )
    def k(tab_hbm, src_hbm, dst_hbm, e2_hbm, zero_hbm, sum_hbm,
          s0, s1, d0, d1, x0, x1, eb, acc, gsem, esem, ssem):
        cid = lax.axis_index("c")
        sid = lax.axis_index("s")
        wid = sid * NC + cid
        pltpu.sync_copy(zero_hbm, acc.at[pl.ds(sid * _RPT, _RPT)])
        plsc.subcore_barrier()
        bufs = ((s0, d0, x0), (s1, d1, x1))

        def e2_load(t):
            base = wid * _RPW + t * _CV
            pltpu.async_copy(e2_hbm.at[pl.ds(base, _CV)], eb, esem)

        def start(t, sb, xb):
            base = wid * _RPW + t * _CV
            pltpu.sync_copy(src_hbm.at[pl.ds(base, _CV)], sb)
            pltpu.async_copy(tab_hbm.at[sb], xb, gsem)

        e2_load(0)
        for b in range(2):
            start(b, bufs[b][0], bufs[b][2])

        def process(t, sb, db, xb, drain, prefetch):
            base = wid * _RPW + t * _CV
            pltpu.make_async_copy(tab_hbm.at[sb], xb, gsem).wait()
            pltpu.make_async_copy(e2_hbm.at[pl.ds(base, _CV)], eb, esem).wait()
            if drain:
                @pl.when(t >= 2)
                def _dr():
                    pltpu.make_async_copy(xb, acc.at[db], ssem).wait()
            pltpu.sync_copy(dst_hbm.at[pl.ds(base, _CV)], db)

            def row(r, carry):
                for c in range(8):
                    sl = pl.ds(c * 16, 16)
                    v = jnp.maximum(xb[r, sl] + eb[r, sl], 0.0) + 1e-7
                    xb[r, sl] = jnp.minimum(v, 100.0)
                return carry

            lax.fori_loop(0, _CV, row, 0)

            @pl.when(t + 1 < _NSV)
            def _el():
                e2_load(t + 1)

            pltpu.async_copy(xb, acc.at[db], ssem, add=True)
            if prefetch:
                @pl.when(t + 2 < _NSV)
                def _pf():
                    start(t + 2, sb, xb)

        def pair(kk, carry):
            for b in range(2):
                t = 2 * kk + b
                process(t, *bufs[b], drain=True, prefetch=True)
            return carry

        lax.fori_loop(0, (_NSV - 1) // 2, pair, 0)
        tl = _NSV - 1
        process(tl, *bufs[tl % 2], drain=True, prefetch=False)
        for t in (_NSV - 2, _NSV - 1):
            sb, db, xb = bufs[t % 2]
            pltpu.make_async_copy(xb, acc.at[db], ssem).wait()
        plsc.subcore_barrier()
        pltpu.sync_copy(acc.at[pl.ds(sid * _RPT, _RPT)],
                        sum_hbm.at[cid, pl.ds(sid * _RPT, _RPT)])

    return k(table, src, dst, e2, zeros)


_CF = 80
_NSF = _RPW // _CF


def _sc_gather_fusion(h, src, dst):
    @functools.partial(
        pl.kernel, mesh=_MESH,
        out_type=[jax.ShapeDtypeStruct((E, 128), jnp.float32),
                  jax.ShapeDtypeStruct((E, 128), jnp.float32)],
        scratch_types=[pltpu.VMEM((_CF,), jnp.int32),
                       pltpu.VMEM((_CF,), jnp.int32),
                       pltpu.VMEM((_CF,), jnp.int32),
                       pltpu.VMEM((_CF,), jnp.int32),
                       pltpu.VMEM((_CF, 128), jnp.float32),
                       pltpu.VMEM((_CF, 128), jnp.float32),
                       pltpu.VMEM((_CF, 128), jnp.float32),
                       pltpu.VMEM((_CF, 128), jnp.float32),
                       pltpu.SemaphoreType.DMA],
    )
    def k(h_hbm, src_hbm, dst_hbm, hs_hbm, hd_hbm,
          s0, s1, d0, d1, hs0, hs1, hd0, hd1, sem):
        wid = lax.axis_index("s") * NC + lax.axis_index("c")
        bufs = ((s0, d0, hs0, hd0), (s1, d1, hs1, hd1))

        def start(t, sb, db, hsb, hdb):
            base = wid * _RPW + t * _CF
            pltpu.sync_copy(src_hbm.at[pl.ds(base, _CF)], sb)
            pltpu.sync_copy(dst_hbm.at[pl.ds(base, _CF)], db)
            pltpu.async_copy(h_hbm.at[sb], hsb, sem)
            pltpu.async_copy(h_hbm.at[db], hdb, sem)

        for b in range(2):
            start(b, *bufs[b])

        def finish(t, sb, db, hsb, hdb):
            pltpu.make_async_copy(h_hbm.at[sb], hsb, sem).wait()
            pltpu.make_async_copy(h_hbm.at[db], hdb, sem).wait()
            base = wid * _RPW + t * _CF
            pltpu.sync_copy(hsb, hs_hbm.at[pl.ds(base, _CF)])
            pltpu.sync_copy(hdb, hd_hbm.at[pl.ds(base, _CF)])

        def pair(kk, carry):
            for b in range(2):
                t = 2 * kk + b
                finish(t, *bufs[b])

                @pl.when(t + 2 < _NSF)
                def _pf():
                    start(t + 2, *bufs[b])
            return carry

        lax.fori_loop(0, (_NSF - 1) // 2, pair, 0)
        tl = _NSF - 1
        finish(tl, *bufs[tl % 2])

    return k(h, src, dst)


def kernel(x, edge_index, time_attr, aux_info, pos, batch, params):
    p = params
    src, dst = edge_index[0], edge_index[1]
    e2_1, e2_2, ew = _edge_pre(time_attr, aux_info, p)
    zeros = jnp.zeros((_RPT, 128), jnp.float32)
    batch_pad = jnp.pad(batch, (0, N_PAD - N), constant_values=G).reshape(N_PAD // 128, 128)
    cnt128, bnd = _counts(dst.reshape(E, 1), batch_pad)
    cnt = cnt128.reshape(N_PAD, 1)[:N]

    sums1 = _sc_conv(x, src, dst, e2_1, zeros)
    h1 = _node_update(x, sums1[0, :N], sums1[1, :N], cnt, p['conv1'])

    sums2 = _sc_conv(h1, src, dst, e2_2, zeros)
    h = _node_update(h1, sums2[0, :N], sums2[1, :N], cnt, p['conv2'])

    hs, hd = _sc_gather_fusion(h, src, dst)
    fmax = _fusion_max(hs, hd, ew, src.reshape(E, 1), bnd)
    fusion = _fusion_head(fmax, p)

    contrastive = _contrastive(h, p)
    recon = _recon(h, p)
    return (h, fusion, contrastive, recon)


# speedup vs baseline: 1.0386x; 1.0004x over previous
import functools

import jax
import jax.numpy as jnp
from jax import lax
from jax.experimental import pallas as pl
from jax.experimental.pallas import tpu as pltpu
from jax.experimental.pallas import tpu_sc as plsc

N = 10000
E = 320000
D = 128
H = 128
EA = 64
TL = 128
G = 16

NC = 2
NS = 16
NW = NC * NS

def _ln(x, g, b, eps=1e-5):
    m = jnp.mean(x, axis=-1, keepdims=True)
    v = jnp.mean((x - m) ** 2, axis=-1, keepdims=True)
    return (x - m) / jnp.sqrt(v + eps) * g + b


_BE = 1280


def _edge_pre_body(ta_ref, ax_ref, tw_ref, tb_ref, aw_ref, ab_ref,
                   eg_ref, eb_ref, w1a_ref, b1a_ref, w1b_ref, b1b_ref,
                   ww1_ref, wb1_ref, wg1_ref, wbt1_ref,
                   ww2_ref, wb2_ref, wg2_ref, wbt2_ref,
                   ww3_ref, wb3_ref,
                   e2a_ref, e2b_ref, ew_ref):
    ta = ta_ref[...]
    ax = ax_ref[...]
    tf = jnp.dot(ta, tw_ref[...], preferred_element_type=jnp.float32) + tb_ref[...]
    af = jnp.dot(ax, aw_ref[...], preferred_element_type=jnp.float32) + ab_ref[...]
    ef = jnp.concatenate([tf, af], axis=-1)
    ef = _ln(ef, eg_ref[...], eb_ref[...])
    e2a_ref[...] = jnp.dot(ef, w1a_ref[...], preferred_element_type=jnp.float32) + b1a_ref[...]
    e2b_ref[...] = jnp.dot(ef, w1b_ref[...], preferred_element_type=jnp.float32) + b1b_ref[...]
    w1 = jax.nn.relu(_ln(jnp.dot(ax, ww1_ref[...], preferred_element_type=jnp.float32) + wb1_ref[...],
                         wg1_ref[...], wbt1_ref[...]))
    w2 = jax.nn.relu(_ln(jnp.dot(w1, ww2_ref[...], preferred_element_type=jnp.float32) + wb2_ref[...],
                         wg2_ref[...], wbt2_ref[...]))
    ew_ref[...] = jax.nn.sigmoid(
        jnp.dot(w2, ww3_ref[...], preferred_element_type=jnp.float32) + wb3_ref[...])


def _full(shape):
    return pl.BlockSpec(shape, lambda i: (0, 0))


def _edge_pre(time_attr, aux_info, p):
    grid = (E // _BE,)
    row = lambda c: pl.BlockSpec((_BE, c), lambda i: (i, 0))
    return pl.pallas_call(
        _edge_pre_body,
        grid=grid,
        in_specs=[
            row(64), row(2),
            _full((64, 32)), _full((1, 32)), _full((2, 32)), _full((1, 32)),
            _full((1, 64)), _full((1, 64)),
            _full((64, 128)), _full((1, 128)), _full((64, 128)), _full((1, 128)),
            _full((2, 128)), _full((1, 128)), _full((1, 128)), _full((1, 128)),
            _full((128, 128)), _full((1, 128)), _full((1, 128)), _full((1, 128)),
            _full((128, 1)), _full((1, 1)),
        ],
        out_specs=[row(128), row(128), row(1)],
        out_shape=[
            jax.ShapeDtypeStruct((E, 128), jnp.float32),
            jax.ShapeDtypeStruct((E, 128), jnp.float32),
            jax.ShapeDtypeStruct((E, 1), jnp.float32),
        ],
    )(time_attr, aux_info,
      p['time_proj_w'], p['time_proj_b'].reshape(1, -1),
      p['aux_proj_w'], p['aux_proj_b'].reshape(1, -1),
      p['edge_norm_g'].reshape(1, -1), p['edge_norm_b'].reshape(1, -1),
      p['conv1']['lin_edge_w'], p['conv1']['lin_edge_b'].reshape(1, -1),
      p['conv2']['lin_edge_w'], p['conv2']['lin_edge_b'].reshape(1, -1),
      p['wn_w1'], p['wn_b1'].reshape(1, -1), p['wn_ln1_g'].reshape(1, -1), p['wn_ln1_b'].reshape(1, -1),
      p['wn_w2'], p['wn_b2'].reshape(1, -1), p['wn_ln2_g'].reshape(1, -1), p['wn_ln2_b'].reshape(1, -1),
      p['wn_w3'], p['wn_b3'].reshape(1, -1))


_BN = 1000


def _node_body(p_ref, sc_ref, x_ref, sa_ref, sb_ref, cn_ref,
               w1_ref, b1_ref, lg_ref, lb_ref, w2_ref, b2_ref, out_ref):
    x = x_ref[...]
    s = sa_ref[...] + sb_ref[...]
    cnt = cn_ref[...]
    mean = s / jnp.maximum(cnt, 1.0)
    mean = jnp.clip(mean, 1e-7, 100.0)
    agg = jnp.exp(jnp.log(mean) / p_ref[0])
    nrm = jnp.sqrt(jnp.sum(agg * agg, axis=-1, keepdims=True))
    mn = agg / jnp.maximum(nrm, 1e-12)
    xn = jnp.sqrt(jnp.sum(x * x, axis=-1, keepdims=True))
    out = x + mn * xn * sc_ref[0]
    t = jnp.dot(out, w1_ref[...], preferred_element_type=jnp.float32) + b1_ref[...]
    t = jax.nn.relu(_ln(t, lg_ref[...], lb_ref[...]))
    out_ref[...] = jnp.dot(t, w2_ref[...], preferred_element_type=jnp.float32) + b2_ref[...]


def _node_update(x, s_a, s_b, cnt, conv):
    row = lambda c: pl.BlockSpec((_BN, c), lambda i: (i, 0))
    return pl.pallas_call(
        _node_body,
        grid=(N // _BN,),
        in_specs=[
            pl.BlockSpec(memory_space=pltpu.SMEM),
            pl.BlockSpec(memory_space=pltpu.SMEM),
            row(128), row(128), row(128), row(1),
            _full((128, 256)), _full((1, 256)), _full((1, 256)), _full((1, 256)),
            _full((256, 128)), _full((1, 128)),
        ],
        out_specs=row(128),
        out_shape=jax.ShapeDtypeStruct((N, 128), jnp.float32),
    )(conv['p'].reshape(1), conv['scale'].reshape(1),
      x, s_a, s_b, cnt,
      conv['mlp_w1'], conv['mlp_b1'].reshape(1, -1),
      conv['mlp_ln_g'].reshape(1, -1), conv['mlp_ln_b'].reshape(1, -1),
      conv['mlp_w2'], conv['mlp_b2'].reshape(1, -1))


def _counts_body(d_ref, b_ref, cnt_ref, bnd_ref):
    @pl.when(pl.program_id(0) == 0)
    def _init():
        cnt_ref[...] = jnp.zeros_like(cnt_ref)
        b = b_ref[...]
        iota = jax.lax.broadcasted_iota(jnp.int32, (1, 128), 1)
        acc = jnp.zeros((1, 128), jnp.int32)
        run = jnp.zeros((), jnp.int32)
        for g in range(G):
            acc = acc + jnp.where(iota == g, run, 0)
            run = run + jnp.sum((b == g).astype(jnp.int32))
        acc = acc + jnp.where(iota == G, run, 0)
        bnd_ref[...] = acc

    d = d_ref[...]
    dlo = (d & 127) == jax.lax.broadcasted_iota(jnp.int32, (1, 128), 1)
    dhi = (d >> 7) == jax.lax.broadcasted_iota(jnp.int32, (1, N_PAD // 128), 1)
    cnt_ref[...] += jax.lax.dot_general(
        dhi.astype(jnp.float32), dlo.astype(jnp.float32),
        (((0,), (0,)), ((), ())), preferred_element_type=jnp.float32)


def _counts(dst, batch_pad):
    return pl.pallas_call(
        _counts_body,
        grid=(E // _BE,),
        in_specs=[pl.BlockSpec((_BE, 1), lambda i: (i, 0)),
                  pl.BlockSpec((N_PAD // 128, 128), lambda i: (0, 0))],
        out_specs=[pl.BlockSpec((N_PAD // 128, 128), lambda i: (0, 0)),
                   pl.BlockSpec((1, 128), lambda i: (0, 0))],
        out_shape=[jax.ShapeDtypeStruct((N_PAD // 128, 128), jnp.float32),
                   jax.ShapeDtypeStruct((1, 128), jnp.int32)],
    )(dst, batch_pad)


_BF = 1280


def _fmax_body(hs_ref, hd_ref, ew_ref, src_ref, bnd_ref, out_ref):
    @pl.when(pl.program_id(0) == 0)
    def _init():
        out_ref[...] = jnp.full_like(out_ref, -jnp.inf)

    w = jnp.concatenate([hs_ref[...], hd_ref[...]], axis=-1) * ew_ref[...]
    s = src_ref[...]
    bnd = bnd_ref[...]
    gid = jnp.zeros_like(s)
    for g in range(1, G):
        gid = gid + (s >= bnd[0, g]).astype(jnp.int32)
    rows = []
    for g in range(G):
        cand = jnp.where(gid == g, w, -jnp.inf)
        rows.append(jnp.max(cand, axis=0, keepdims=True))
    out_ref[...] = jnp.maximum(out_ref[...], jnp.concatenate(rows, axis=0))


def _fusion_max(hs, hd, ew, src, bnd):
    row = lambda c: pl.BlockSpec((_BF, c), lambda i: (i, 0))
    return pl.pallas_call(
        _fmax_body,
        grid=(E // _BF,),
        in_specs=[row(128), row(128), row(1), row(1),
                  pl.BlockSpec((1, 128), lambda i: (0, 0))],
        out_specs=pl.BlockSpec((G, 256), lambda i: (0, 0)),
        out_shape=jax.ShapeDtypeStruct((G, 256), jnp.float32),
    )(hs, hd, ew, src, bnd)


def _fhead_body(f_ref, g_ref, b_ref, w_ref, wb_ref, out_ref):
    t = _ln(f_ref[...], g_ref[...], b_ref[...])
    out_ref[...] = jax.nn.sigmoid(
        jnp.dot(t, w_ref[...], preferred_element_type=jnp.float32) + wb_ref[...])


def _fusion_head(fmax, p):
    return pl.pallas_call(
        _fhead_body,
        grid=(1,),
        in_specs=[_full((G, 256)),
                  _full((1, 256)), _full((1, 256)),
                  _full((256, TL)), _full((1, TL))],
        out_specs=_full((G, TL)),
        out_shape=jax.ShapeDtypeStruct((G, TL), jnp.float32),
    )(fmax, p['norm_g'].reshape(1, -1), p['norm_b'].reshape(1, -1),
      p['fuse_w'], p['fuse_b'].reshape(1, -1))


def _contr_a_body(h_ref, w_ref, b_ref, y_ref, st_ref):
    @pl.when(pl.program_id(0) == 0)
    def _init():
        st_ref[...] = jnp.zeros_like(st_ref)

    y = jnp.dot(h_ref[...], w_ref[...], preferred_element_type=jnp.float32) + b_ref[...]
    y_ref[...] = y
    st_ref[0:1, :] += jnp.sum(y, axis=0, keepdims=True)
    st_ref[1:2, :] += jnp.sum(y * y, axis=0, keepdims=True)


def _contr_b_body(y_ref, st_ref, g_ref, b_ref, w_ref, wb_ref, out_ref):
    mean = st_ref[0:1, :] / N
    var = st_ref[1:2, :] / N - mean * mean
    xh = (y_ref[...] - mean) / jnp.sqrt(var + 1e-5) * g_ref[...] + b_ref[...]
    xh = jax.nn.relu(xh)
    out_ref[...] = jnp.dot(xh, w_ref[...], preferred_element_type=jnp.float32) + wb_ref[...]


def _contrastive(h, p):
    row = lambda c: pl.BlockSpec((_BN, c), lambda i: (i, 0))
    y, st = pl.pallas_call(
        _contr_a_body,
        grid=(N // _BN,),
        in_specs=[row(128), _full((128, 128)), _full((1, 128))],
        out_specs=[row(128), pl.BlockSpec((8, 128), lambda i: (0, 0))],
        out_shape=[jax.ShapeDtypeStruct((N, 128), jnp.float32),
                   jax.ShapeDtypeStruct((8, 128), jnp.float32)],
    )(h, p['c_w1'], p['c_b1'].reshape(1, -1))
    return pl.pallas_call(
        _contr_b_body,
        grid=(N // _BN,),
        in_specs=[row(128), _full((8, 128)), _full((1, 128)), _full((1, 128)),
                  _full((128, 128)), _full((1, 128))],
        out_specs=row(128),
        out_shape=jax.ShapeDtypeStruct((N, 128), jnp.float32),
    )(y, st, p['c_bn_g'].reshape(1, -1), p['c_bn_b'].reshape(1, -1),
      p['c_w2'], p['c_b2'].reshape(1, -1))


def _recon_body(h_ref, w1_ref, b1_ref, w2_ref, b2_ref,
                w3_ref, b3_ref, w4_ref, b4_ref, out_ref):
    t = jax.nn.relu(jnp.dot(h_ref[...], w1_ref[...], preferred_element_type=jnp.float32) + b1_ref[...])
    enc = jnp.dot(t, w2_ref[...], preferred_element_type=jnp.float32) + b2_ref[...]
    t2 = jax.nn.relu(jnp.dot(enc, w3_ref[...], preferred_element_type=jnp.float32) + b3_ref[...])
    out_ref[...] = jnp.dot(t2, w4_ref[...], preferred_element_type=jnp.float32) + b4_ref[...]


def _recon(h, p):
    row = lambda c: pl.BlockSpec((_BN, c), lambda i: (i, 0))
    return pl.pallas_call(
        _recon_body,
        grid=(N // _BN,),
        in_specs=[row(128),
                  _full((128, 128)), _full((1, 128)),
                  _full((128, 64)), _full((1, 64)),
                  _full((64, 128)), _full((1, 128)),
                  _full((128, 128)), _full((1, 128))],
        out_specs=row(128),
        out_shape=jax.ShapeDtypeStruct((N, 128), jnp.float32),
    )(h, p['r_e_w1'], p['r_e_b1'].reshape(1, -1),
      p['r_e_w2'], p['r_e_b2'].reshape(1, -1),
      p['r_d_w1'], p['r_d_b1'].reshape(1, -1),
      p['r_d_w2'], p['r_d_b2'].reshape(1, -1))


_MESH = plsc.VectorSubcoreMesh(core_axis_name="c", subcore_axis_name="s",
                               num_cores=NC, num_subcores=NS)
_C = 80
_RPW = E // NW
_NST = _RPW // _C
N_PAD = 10240
_RPT = N_PAD // NS


_CV = 40
_NSV = _RPW // _CV


def _sc_conv(table, src, dst, e2, zeros):
    @functools.partial(
        pl.kernel, mesh=_MESH,
        out_type=jax.ShapeDtypeStruct((NC, N_PAD, 128), jnp.float32),
        scratch_types=[pltpu.VMEM((_CV,), jnp.int32),
                       pltpu.VMEM((_CV,), jnp.int32),
                       pltpu.VMEM((_CV,), jnp.int32),
                       pltpu.VMEM((_CV,), jnp.int32),
                       pltpu.VMEM((_CV, 128), jnp.float32),
                       pltpu.VMEM((_CV, 128), jnp.float32),
                       pltpu.VMEM((_CV, 128), jnp.float32),
                       pltpu.VMEM((_CV, 128), jnp.float32),
                       pltpu.VMEM_SHARED((N_PAD, 128), jnp.float32),
                       pltpu.SemaphoreType.DMA,
                       pltpu.SemaphoreType.DMA,
                       pltpu.SemaphoreType.DMA],
    )
    def k(tab_hbm, src_hbm, dst_hbm, e2_hbm, zero_hbm, sum_hbm,
          s0, s1, d0, d1, x0, x1, e0, e1, acc, gsem, esem, ssem):
        cid = lax.axis_index("c")
        sid = lax.axis_index("s")
        wid = sid * NC + cid
        pltpu.sync_copy(zero_hbm, acc.at[pl.ds(sid * _RPT, _RPT)])
        plsc.subcore_barrier()
        bufs = ((s0, d0, x0, e0), (s1, d1, x1, e1))

        def start(t, sb, xb, eb):
            base = wid * _RPW + t * _CV
            pltpu.sync_copy(src_hbm.at[pl.ds(base, _CV)], sb)
            pltpu.async_copy(e2_hbm.at[pl.ds(base, _CV)], eb, esem)
            pltpu.async_copy(tab_hbm.at[sb], xb, gsem)

        for b in range(2):
            start(b, bufs[b][0], bufs[b][2], bufs[b][3])

        def process(t, sb, db, xb, eb, drain, prefetch):
            base = wid * _RPW + t * _CV
            pltpu.make_async_copy(tab_hbm.at[sb], xb, gsem).wait()
            pltpu.make_async_copy(e2_hbm.at[pl.ds(base, _CV)], eb, esem).wait()
            if drain:
                @pl.when(t >= 2)
                def _dr():
                    pltpu.make_async_copy(xb, acc.at[db], ssem).wait()
            pltpu.sync_copy(dst_hbm.at[pl.ds(base, _CV)], db)

            def row(r, carry):
                for c in range(8):
                    sl = pl.ds(c * 16, 16)
                    v = jnp.maximum(xb[r, sl] + eb[r, sl], 0.0) + 1e-7
                    xb[r, sl] = jnp.minimum(v, 100.0)
                return carry

            lax.fori_loop(0, _CV, row, 0)
            pltpu.async_copy(xb, acc.at[db], ssem, add=True)
            if prefetch:
                @pl.when(t + 2 < _NSV)
                def _pf():
                    start(t + 2, sb, xb, eb)

        def pair(kk, carry):
            for b in range(2):
                t = 2 * kk + b
                process(t, *bufs[b], drain=True, prefetch=True)
            return carry

        lax.fori_loop(0, _NSV // 2 - 1, pair, 0)
        for t in (_NSV - 2, _NSV - 1):
            process(t, *bufs[t % 2], drain=True, prefetch=False)
        for t in (_NSV - 2, _NSV - 1):
            sb, db, xb, eb = bufs[t % 2]
            pltpu.make_async_copy(xb, acc.at[db], ssem).wait()
        plsc.subcore_barrier()
        pltpu.sync_copy(acc.at[pl.ds(sid * _RPT, _RPT)],
                        sum_hbm.at[cid, pl.ds(sid * _RPT, _RPT)])

    return k(table, src, dst, e2, zeros)


def _counts_body(d_ref, b_ref, cnt_ref, bnd_ref):
    @pl.when(pl.program_id(0) == 0)
    def _init():
        cnt_ref[...] = jnp.zeros_like(cnt_ref)
        b = b_ref[...]
        iota = jax.lax.broadcasted_iota(jnp.int32, (1, 128), 1)
        acc = jnp.zeros((1, 128), jnp.int32)
        run = jnp.zeros((), jnp.int32)
        for g in range(G):
            acc = acc + jnp.where(iota == g, run, 0)
            run = run + jnp.sum((b == g).astype(jnp.int32))
        acc = acc + jnp.where(iota == G, run, 0)
        bnd_ref[...] = acc

    d = d_ref[...]
    dlo = (d & 127) == jax.lax.broadcasted_iota(jnp.int32, (1, 128), 1)
    dhi = (d >> 7) == jax.lax.broadcasted_iota(jnp.int32, (1, N_PAD // 128), 1)
    cnt_ref[...] += jax.lax.dot_general(
        dhi.astype(jnp.float32), dlo.astype(jnp.float32),
        (((0,), (0,)), ((), ())), preferred_element_type=jnp.float32)


def _counts(dst, batch_pad):
    return pl.pallas_call(
        _counts_body,
        grid=(E // _BE,),
        in_specs=[pl.BlockSpec((_BE, 1), lambda i: (i, 0)),
                  pl.BlockSpec((N_PAD // 128, 128), lambda i: (0, 0))],
        out_specs=[pl.BlockSpec((N_PAD // 128, 128), lambda i: (0, 0)),
                   pl.BlockSpec((1, 128), lambda i: (0, 0))],
        out_shape=[jax.ShapeDtypeStruct((N_PAD // 128, 128), jnp.float32),
                   jax.ShapeDtypeStruct((1, 128), jnp.int32)],
    )(dst, batch_pad)


_BF = 1280


def _fmax_body(hs_ref, hd_ref, ew_ref, src_ref, bnd_ref, out_ref):
    @pl.when(pl.program_id(0) == 0)
    def _init():
        out_ref[...] = jnp.full_like(out_ref, -jnp.inf)

    w = jnp.concatenate([hs_ref[...], hd_ref[...]], axis=-1) * ew_ref[...]
    s = src_ref[...]
    bnd = bnd_ref[...]
    gid = jnp.zeros_like(s)
    for g in range(1, G):
        gid = gid + (s >= bnd[0, g]).astype(jnp.int32)
    rows = []
    for g in range(G):
        cand = jnp.where(gid == g, w, -jnp.inf)
        rows.append(jnp.max(cand, axis=0, keepdims=True))
    out_ref[...] = jnp.maximum(out_ref[...], jnp.concatenate(rows, axis=0))


def _fusion_max(hs, hd, ew, src, bnd):
    row = lambda c: pl.BlockSpec((_BF, c), lambda i: (i, 0))
    return pl.pallas_call(
        _fmax_body,
        grid=(E // _BF,),
        in_specs=[row(128), row(128), row(1), row(1),
                  pl.BlockSpec((1, 128), lambda i: (0, 0))],
        out_specs=pl.BlockSpec((G, 256), lambda i: (0, 0)),
        out_shape=jax.ShapeDtypeStruct((G, 256), jnp.float32),
    )(hs, hd, ew, src, bnd)


def _fhead_body(f_ref, g_ref, b_ref, w_ref, wb_ref, out_ref):
    t = _ln(f_ref[...], g_ref[...], b_ref[...])
    out_ref[...] = jax.nn.sigmoid(
        jnp.dot(t, w_ref[...], preferred_element_type=jnp.float32) + wb_ref[...])


def _fusion_head(fmax, p):
    return pl.pallas_call(
        _fhead_body,
        grid=(1,),
        in_specs=[_full((G, 256)),
                  _full((1, 256)), _full((1, 256)),
                  _full((256, TL)), _full((1, TL))],
        out_specs=_full((G, TL)),
        out_shape=jax.ShapeDtypeStruct((G, TL), jnp.float32),
    )(fmax, p['norm_g'].reshape(1, -1), p['norm_b'].reshape(1, -1),
      p['fuse_w'], p['fuse_b'].reshape(1, -1))


def _contr_a_body(h_ref, w_ref, b_ref, y_ref, st_ref):
    @pl.when(pl.program_id(0) == 0)
    def _init():
        st_ref[...] = jnp.zeros_like(st_ref)

    y = jnp.dot(h_ref[...], w_ref[...], preferred_element_type=jnp.float32) + b_ref[...]
    y_ref[...] = y
    st_ref[0:1, :] += jnp.sum(y, axis=0, keepdims=True)
    st_ref[1:2, :] += jnp.sum(y * y, axis=0, keepdims=True)


def _contr_b_body(y_ref, st_ref, g_ref, b_ref, w_ref, wb_ref, out_ref):
    mean = st_ref[0:1, :] / N
    var = st_ref[1:2, :] / N - mean * mean
    xh = (y_ref[...] - mean) / jnp.sqrt(var + 1e-5) * g_ref[...] + b_ref[...]
    xh = jax.nn.relu(xh)
    out_ref[...] = jnp.dot(xh, w_ref[...], preferred_element_type=jnp.float32) + wb_ref[...]


def _contrastive(h, p):
    row = lambda c: pl.BlockSpec((_BN, c), lambda i: (i, 0))
    y, st = pl.pallas_call(
        _contr_a_body,
        grid=(N // _BN,),
        in_specs=[row(128), _full((128, 128)), _full((1, 128))],
        out_specs=[row(128), pl.BlockSpec((8, 128), lambda i: (0, 0))],
        out_shape=[jax.ShapeDtypeStruct((N, 128), jnp.float32),
                   jax.ShapeDtypeStruct((8, 128), jnp.float32)],
    )(h, p['c_w1'], p['c_b1'].reshape(1, -1))
    return pl.pallas_call(
        _contr_b_body,
        grid=(N // _BN,),
        in_specs=[row(128), _full((8, 128)), _full((1, 128)), _full((1, 128)),
                  _full((128, 128)), _full((1, 128))],
        out_specs=row(128),
        out_shape=jax.ShapeDtypeStruct((N, 128), jnp.float32),
    )(y, st, p['c_bn_g'].reshape(1, -1), p['c_bn_b'].reshape(1, -1),
      p['c_w2'], p['c_b2'].reshape(1, -1))


def _recon_body(h_ref, w1_ref, b1_ref, w2_ref, b2_ref,
                w3_ref, b3_ref, w4_ref, b4_ref, out_ref):
    t = jax.nn.relu(jnp.dot(h_ref[...], w1_ref[...], preferred_element_type=jnp.float32) + b1_ref[...])
    enc = jnp.dot(t, w2_ref[...], preferred_element_type=jnp.float32) + b2_ref[...]
    t2 = jax.nn.relu(jnp.dot(enc, w3_ref[...], preferred_element_type=jnp.float32) + b3_ref[...])
    out_ref[...] = jnp.dot(t2, w4_ref[...], preferred_element_type=jnp.float32) + b4_ref[...]


def _recon(h, p):
    row = lambda c: pl.BlockSpec((_BN, c), lambda i: (i, 0))
    return pl.pallas_call(
        _recon_body,
        grid=(N // _BN,),
        in_specs=[row(128),
                  _full((128, 128)), _full((1, 128)),
                  _full((128, 64)), _full((1, 64)),
                  _full((64, 128)), _full((1, 128)),
                  _full((128, 128)), _full((1, 128))],
        out_specs=row(128),
        out_shape=jax.ShapeDtypeStruct((N, 128), jnp.float32),
    )(h, p['r_e_w1'], p['r_e_b1'].reshape(1, -1),
      p['r_e_w2'], p['r_e_b2'].reshape(1, -1),
      p['r_d_w1'], p['r_d_b1'].reshape(1, -1),
      p['r_d_w2'], p['r_d_b2'].reshape(1, -1))


_MESH = plsc.VectorSubcoreMesh(core_axis_name="c", subcore_axis_name="s",
                               num_cores=NC, num_subcores=NS)
_C = 80
_RPW = E // NW
_NST = _RPW // _C
N_PAD = 10240
_RPT = N_PAD // NS


_CV = 80
_NSV = _RPW // _CV


def _sc_conv(table, src, dst, e2, zeros):
    @functools.partial(
        pl.kernel, mesh=_MESH,
        out_type=jax.ShapeDtypeStruct((NC, N_PAD, 128), jnp.float32),
        scratch_types=[pltpu.VMEM((_CV,), jnp.int32),
                       pltpu.VMEM((_CV,), jnp.int32),
                       pltpu.VMEM((_CV,), jnp.int32),
                       pltpu.VMEM((_CV,), jnp.int32),
                       pltpu.VMEM((_CV, 128), jnp.float32),
                       pltpu.VMEM((_CV, 128), jnp.float32),
                       pltpu.VMEM((_CV, 128), jnp.float32),
                       pltpu.VMEM_SHARED((N_PAD, 128), jnp.float32),
                       pltpu.SemaphoreType.DMA,
                       pltpu.SemaphoreType.DMA,
                       pltpu.SemaphoreType.DMA],
    )
    def k(tab_hbm, src_hbm, dst_hbm, e2_hbm, zero_hbm, sum_hbm,
          s0, s1, d0, d1, x0, x1, eb, acc, gsem, esem, ssem):
        cid = lax.axis_index("c")
        sid = lax.axis_index("s")
        wid = sid * NC + cid
        pltpu.sync_copy(zero_hbm, acc.at[pl.ds(sid * _RPT, _RPT)])
        plsc.subcore_barrier()
        bufs = ((s0, d0, x0), (s1, d1, x1))

        def e2_load(t):
            base = wid * _RPW + t * _CV
            pltpu.async_copy(e2_hbm.at[pl.ds(base, _CV)], eb, esem)

        def start(t, sb, xb):
            base = wid * _RPW + t * _CV
            pltpu.sync_copy(src_hbm.at[pl.ds(base, _CV)], sb)
            pltpu.async_copy(tab_hbm.at[sb], xb, gsem)

        e2_load(0)
        for b in range(2):
            start(b, bufs[b][0], bufs[b][2])

        def process(t, sb, db, xb, drain, prefetch):
            base = wid * _RPW + t * _CV
            pltpu.make_async_copy(tab_hbm.at[sb], xb, gsem).wait()
            pltpu.make_async_copy(e2_hbm.at[pl.ds(base, _CV)], eb, esem).wait()
            if drain:
                @pl.when(t >= 2)
                def _dr():
                    pltpu.make_async_copy(xb, acc.at[db], ssem).wait()
            pltpu.sync_copy(dst_hbm.at[pl.ds(base, _CV)], db)

            def row(r, carry):
                for c in range(8):
                    sl = pl.ds(c * 16, 16)
                    v = jnp.maximum(xb[r, sl] + eb[r, sl], 0.0) + 1e-7
                    xb[r, sl] = jnp.minimum(v, 100.0)
                return carry

            lax.fori_loop(0, _CV, row, 0)

            @pl.when(t + 1 < _NSV)
            def _el():
                e2_load(t + 1)

            pltpu.async_copy(xb, acc.at[db], ssem, add=True)
            if prefetch:
                @pl.when(t + 2 < _NSV)
                def _pf():
                    start(t + 2, sb, xb)

        def pair(kk, carry):
            for b in range(2):
                t = 2 * kk + b
                process(t, *bufs[b], drain=True, prefetch=True)
            return carry

        lax.fori_loop(0, (_NSV - 1) // 2, pair, 0)
        tl = _NSV - 1
        process(tl, *bufs[tl % 2], drain=True, prefetch=False)
        for t in (_NSV - 2, _NSV - 1):
            sb, db, xb = bufs[t % 2]
            pltpu.make_async_copy(xb, acc.at[db], ssem).wait()
        plsc.subcore_barrier()
        pltpu.sync_copy(acc.at[pl.ds(sid * _RPT, _RPT)],
                        sum_hbm.at[cid, pl.ds(sid * _RPT, _RPT)])

    return k(table, src, dst, e2, zeros)


_CF = 80
_NSF = _RPW // _CF


def _sc_gather_fusion(h, src, dst):
    @functools.partial(
        pl.kernel, mesh=_MESH,
        out_type=[jax.ShapeDtypeStruct((E, 128), jnp.float32),
                  jax.ShapeDtypeStruct((E, 128), jnp.float32)],
        scratch_types=[pltpu.VMEM((_CF,), jnp.int32),
                       pltpu.VMEM((_CF,), jnp.int32),
                       pltpu.VMEM((_CF,), jnp.int32),
                       pltpu.VMEM((_CF,), jnp.int32),
                       pltpu.VMEM((_CF, 128), jnp.float32),
                       pltpu.VMEM((_CF, 128), jnp.float32),
                       pltpu.VMEM((_CF, 128), jnp.float32),
                       pltpu.VMEM((_CF, 128), jnp.float32),
                       pltpu.SemaphoreType.DMA],
    )
    def k(h_hbm, src_hbm, dst_hbm, hs_hbm, hd_hbm,
          s0, s1, d0, d1, hs0, hs1, hd0, hd1, sem):
        wid = lax.axis_index("s") * NC + lax.axis_index("c")
        bufs = ((s0, d0, hs0, hd0), (s1, d1, hs1, hd1))

        def start(t, sb, db, hsb, hdb):
            base = wid * _RPW + t * _CF
            pltpu.sync_copy(src_hbm.at[pl.ds(base, _CF)], sb)
            pltpu.sync_copy(dst_hbm.at[pl.ds(base, _CF)], db)
            pltpu.async_copy(h_hbm.at[sb], hsb, sem)
            pltpu.async_copy(h_hbm.at[db], hdb, sem)

        for b in range(2):
            start(b, *bufs[b])

        def finish(t, sb, db, hsb, hdb):
            pltpu.make_async_copy(h_hbm.at[sb], hsb, sem).wait()
            pltpu.make_async_copy(h_hbm.at[db], hdb, sem).wait()
            base = wid * _RPW + t * _CF
            pltpu.sync_copy(hsb, hs_hbm.at[pl.ds(base, _CF)])
            pltpu.sync_copy(hdb, hd_hbm.at[pl.ds(base, _CF)])

        def pair(kk, carry):
            for b in range(2):
                t = 2 * kk + b
                finish(t, *bufs[b])

                @pl.when(t + 2 < _NSF)
                def _pf():
                    start(t + 2, *bufs[b])
            return carry

        lax.fori_loop(0, (_NSF - 1) // 2, pair, 0)
        tl = _NSF - 1
        finish(tl, *bufs[tl % 2])

    return k(h, src, dst)


def kernel(x, edge_index, time_attr, aux_info, pos, batch, params):
    p = params
    src, dst = edge_index[0], edge_index[1]
    e2_1, e2_2, ew = _edge_pre(time_attr, aux_info, p)
    zeros = jnp.zeros((_RPT, 128), jnp.float32)
    batch_pad = jnp.pad(batch, (0, N_PAD - N), constant_values=G).reshape(N_PAD // 128, 128)
    cnt128, bnd = _counts(dst.reshape(E, 1), batch_pad)
    cnt = cnt128.reshape(N_PAD, 1)[:N]

    sums1 = _sc_conv(x, src, dst, e2_1, zeros)
    h1 = _node_update(x, sums1[0, :N], sums1[1, :N], cnt, p['conv1'])

    sums2 = _sc_conv(h1, src, dst, e2_2, zeros)
    h = _node_update(h1, sums2[0, :N], sums2[1, :N], cnt, p['conv2'])

    hs, hd = _sc_gather_fusion(h, src, dst)
    fmax = _fusion_max(hs, hd, ew, src.reshape(E, 1), bnd)
    fusion = _fusion_head(fmax, p)

    contrastive = _contrastive(h, p)
    recon = _recon(h, p)
    return (h, fusion, contrastive, recon)


# speedup vs baseline: 1.1352x; 1.0930x over previous
import functools

import jax
import jax.numpy as jnp
from jax import lax
from jax.experimental import pallas as pl
from jax.experimental.pallas import tpu as pltpu
from jax.experimental.pallas import tpu_sc as plsc

N = 10000
E = 320000
D = 128
H = 128
EA = 64
TL = 128
G = 16

NC = 2
NS = 16
NW = NC * NS

def _ln(x, g, b, eps=1e-5):
    m = jnp.mean(x, axis=-1, keepdims=True)
    v = jnp.mean((x - m) ** 2, axis=-1, keepdims=True)
    return (x - m) / jnp.sqrt(v + eps) * g + b


_BE = 1280


def _edge_pre_body(ta_ref, ax_ref, tw_ref, tb_ref, aw_ref, ab_ref,
                   eg_ref, eb_ref, w1a_ref, b1a_ref, w1b_ref, b1b_ref,
                   ww1_ref, wb1_ref, wg1_ref, wbt1_ref,
                   ww2_ref, wb2_ref, wg2_ref, wbt2_ref,
                   ww3_ref, wb3_ref,
                   e2a_ref, e2b_ref, ew_ref):
    ta = ta_ref[...]
    ax = ax_ref[...]
    tf = jnp.dot(ta, tw_ref[...], preferred_element_type=jnp.float32) + tb_ref[...]
    af = jnp.dot(ax, aw_ref[...], preferred_element_type=jnp.float32) + ab_ref[...]
    ef = jnp.concatenate([tf, af], axis=-1)
    ef = _ln(ef, eg_ref[...], eb_ref[...])
    e2a_ref[...] = jnp.dot(ef, w1a_ref[...], preferred_element_type=jnp.float32) + b1a_ref[...]
    e2b_ref[...] = jnp.dot(ef, w1b_ref[...], preferred_element_type=jnp.float32) + b1b_ref[...]
    w1 = jax.nn.relu(_ln(jnp.dot(ax, ww1_ref[...], preferred_element_type=jnp.float32) + wb1_ref[...],
                         wg1_ref[...], wbt1_ref[...]))
    w2 = jax.nn.relu(_ln(jnp.dot(w1, ww2_ref[...], preferred_element_type=jnp.float32) + wb2_ref[...],
                         wg2_ref[...], wbt2_ref[...]))
    ew_ref[...] = jax.nn.sigmoid(
        jnp.dot(w2, ww3_ref[...], preferred_element_type=jnp.float32) + wb3_ref[...])


def _full(shape):
    return pl.BlockSpec(shape, lambda i: (0, 0))


def _edge_pre(time_attr, aux_info, p):
    grid = (E // _BE,)
    row = lambda c: pl.BlockSpec((_BE, c), lambda i: (i, 0))
    return pl.pallas_call(
        _edge_pre_body,
        grid=grid,
        in_specs=[
            row(64), row(2),
            _full((64, 32)), _full((1, 32)), _full((2, 32)), _full((1, 32)),
            _full((1, 64)), _full((1, 64)),
            _full((64, 128)), _full((1, 128)), _full((64, 128)), _full((1, 128)),
            _full((2, 128)), _full((1, 128)), _full((1, 128)), _full((1, 128)),
            _full((128, 128)), _full((1, 128)), _full((1, 128)), _full((1, 128)),
            _full((128, 1)), _full((1, 1)),
        ],
        out_specs=[row(128), row(128), row(1)],
        out_shape=[
            jax.ShapeDtypeStruct((E, 128), jnp.float32),
            jax.ShapeDtypeStruct((E, 128), jnp.float32),
            jax.ShapeDtypeStruct((E, 1), jnp.float32),
        ],
    )(time_attr, aux_info,
      p['time_proj_w'], p['time_proj_b'].reshape(1, -1),
      p['aux_proj_w'], p['aux_proj_b'].reshape(1, -1),
      p['edge_norm_g'].reshape(1, -1), p['edge_norm_b'].reshape(1, -1),
      p['conv1']['lin_edge_w'], p['conv1']['lin_edge_b'].reshape(1, -1),
      p['conv2']['lin_edge_w'], p['conv2']['lin_edge_b'].reshape(1, -1),
      p['wn_w1'], p['wn_b1'].reshape(1, -1), p['wn_ln1_g'].reshape(1, -1), p['wn_ln1_b'].reshape(1, -1),
      p['wn_w2'], p['wn_b2'].reshape(1, -1), p['wn_ln2_g'].reshape(1, -1), p['wn_ln2_b'].reshape(1, -1),
      p['wn_w3'], p['wn_b3'].reshape(1, -1))


_BN = 1000


def _node_body(p_ref, sc_ref, x_ref, sa_ref, sb_ref, cn_ref,
               w1_ref, b1_ref, lg_ref, lb_ref, w2_ref, b2_ref, out_ref):
    x = x_ref[...]
    s = sa_ref[...] + sb_ref[...]
    cnt = cn_ref[...]
    mean = s / jnp.maximum(cnt, 1.0)
    mean = jnp.clip(mean, 1e-7, 100.0)
    agg = jnp.exp(jnp.log(mean) / p_ref[0])
    nrm = jnp.sqrt(jnp.sum(agg * agg, axis=-1, keepdims=True))
    mn = agg / jnp.maximum(nrm, 1e-12)
    xn = jnp.sqrt(jnp.sum(x * x, axis=-1, keepdims=True))
    out = x + mn * xn * sc_ref[0]
    t = jnp.dot(out, w1_ref[...], preferred_element_type=jnp.float32) + b1_ref[...]
    t = jax.nn.relu(_ln(t, lg_ref[...], lb_ref[...]))
    out_ref[...] = jnp.dot(t, w2_ref[...], preferred_element_type=jnp.float32) + b2_ref[...]


def _node_update(x, s_a, s_b, cnt, conv):
    row = lambda c: pl.BlockSpec((_BN, c), lambda i: (i, 0))
    return pl.pallas_call(
        _node_body,
        grid=(N // _BN,),
        in_specs=[
            pl.BlockSpec(memory_space=pltpu.SMEM),
            pl.BlockSpec(memory_space=pltpu.SMEM),
            row(128), row(128), row(128), row(1),
            _full((128, 256)), _full((1, 256)), _full((1, 256)), _full((1, 256)),
            _full((256, 128)), _full((1, 128)),
        ],
        out_specs=row(128),
        out_shape=jax.ShapeDtypeStruct((N, 128), jnp.float32),
    )(conv['p'].reshape(1), conv['scale'].reshape(1),
      x, s_a, s_b, cnt,
      conv['mlp_w1'], conv['mlp_b1'].reshape(1, -1),
      conv['mlp_ln_g'].reshape(1, -1), conv['mlp_ln_b'].reshape(1, -1),
      conv['mlp_w2'], conv['mlp_b2'].reshape(1, -1))


def _counts_body(d_ref, out_ref):
    @pl.when(pl.program_id(0) == 0)
    def _init():
        out_ref[...] = jnp.zeros_like(out_ref)

    d = d_ref[...]
    lo = (d & 127) == jax.lax.broadcasted_iota(jnp.int32, (1, 128), 1)
    hi = (d >> 7) == jax.lax.broadcasted_iota(jnp.int32, (1, N_PAD // 128), 1)
    out_ref[...] += jax.lax.dot_general(
        hi.astype(jnp.float32), lo.astype(jnp.float32),
        (((0,), (0,)), ((), ())), preferred_element_type=jnp.float32)


def _counts(dst):
    return pl.pallas_call(
        _counts_body,
        grid=(E // _BE,),
        in_specs=[pl.BlockSpec((_BE, 1), lambda i: (i, 0))],
        out_specs=pl.BlockSpec((N_PAD // 128, 128), lambda i: (0, 0)),
        out_shape=jax.ShapeDtypeStruct((N_PAD // 128, 128), jnp.float32),
    )(dst)


def _fmax_body(hs_ref, hd_ref, ew_ref, src_ref, b_ref, out_ref, bnd_ref):
    @pl.when(pl.program_id(0) == 0)
    def _init():
        out_ref[...] = jnp.full_like(out_ref, -jnp.inf)
        b = b_ref[...]
        bnd_ref[0] = 0
        for g in range(1, G):
            bnd_ref[g] = jnp.sum((b < g).astype(jnp.int32))
        bnd_ref[G] = N

    w = jnp.concatenate([hs_ref[...], hd_ref[...]], axis=-1) * ew_ref[...]
    s = src_ref[...]
    for g in range(G):
        mask = (s >= bnd_ref[g]) & (s < bnd_ref[g + 1])
        cand = jnp.where(mask, w, -jnp.inf)
        m = jnp.max(cand, axis=0, keepdims=True)
        out_ref[g:g + 1, :] = jnp.maximum(out_ref[g:g + 1, :], m)


def _fusion_max(hs, hd, ew, src, batch_pad):
    row = lambda c: pl.BlockSpec((_BE, c), lambda i: (i, 0))
    return pl.pallas_call(
        _fmax_body,
        grid=(E // _BE,),
        in_specs=[row(128), row(128), row(1), row(1),
                  pl.BlockSpec((N_PAD // 128, 128), lambda i: (0, 0))],
        out_specs=pl.BlockSpec((G, 256), lambda i: (0, 0)),
        out_shape=jax.ShapeDtypeStruct((G, 256), jnp.float32),
        scratch_shapes=[pltpu.SMEM((G + 1,), jnp.int32)],
    )(hs, hd, ew, src, batch_pad)


def _fhead_body(f_ref, g_ref, b_ref, w_ref, wb_ref, out_ref):
    t = _ln(f_ref[...], g_ref[...], b_ref[...])
    out_ref[...] = jax.nn.sigmoid(
        jnp.dot(t, w_ref[...], preferred_element_type=jnp.float32) + wb_ref[...])


def _fusion_head(fmax, p):
    return pl.pallas_call(
        _fhead_body,
        grid=(1,),
        in_specs=[_full((G, 256)),
                  _full((1, 256)), _full((1, 256)),
                  _full((256, TL)), _full((1, TL))],
        out_specs=_full((G, TL)),
        out_shape=jax.ShapeDtypeStruct((G, TL), jnp.float32),
    )(fmax, p['norm_g'].reshape(1, -1), p['norm_b'].reshape(1, -1),
      p['fuse_w'], p['fuse_b'].reshape(1, -1))


def _contr_a_body(h_ref, w_ref, b_ref, y_ref, st_ref):
    @pl.when(pl.program_id(0) == 0)
    def _init():
        st_ref[...] = jnp.zeros_like(st_ref)

    y = jnp.dot(h_ref[...], w_ref[...], preferred_element_type=jnp.float32) + b_ref[...]
    y_ref[...] = y
    st_ref[0:1, :] += jnp.sum(y, axis=0, keepdims=True)
    st_ref[1:2, :] += jnp.sum(y * y, axis=0, keepdims=True)


def _contr_b_body(y_ref, st_ref, g_ref, b_ref, w_ref, wb_ref, out_ref):
    mean = st_ref[0:1, :] / N
    var = st_ref[1:2, :] / N - mean * mean
    xh = (y_ref[...] - mean) / jnp.sqrt(var + 1e-5) * g_ref[...] + b_ref[...]
    xh = jax.nn.relu(xh)
    out_ref[...] = jnp.dot(xh, w_ref[...], preferred_element_type=jnp.float32) + wb_ref[...]


def _contrastive(h, p):
    row = lambda c: pl.BlockSpec((_BN, c), lambda i: (i, 0))
    y, st = pl.pallas_call(
        _contr_a_body,
        grid=(N // _BN,),
        in_specs=[row(128), _full((128, 128)), _full((1, 128))],
        out_specs=[row(128), pl.BlockSpec((8, 128), lambda i: (0, 0))],
        out_shape=[jax.ShapeDtypeStruct((N, 128), jnp.float32),
                   jax.ShapeDtypeStruct((8, 128), jnp.float32)],
    )(h, p['c_w1'], p['c_b1'].reshape(1, -1))
    return pl.pallas_call(
        _contr_b_body,
        grid=(N // _BN,),
        in_specs=[row(128), _full((8, 128)), _full((1, 128)), _full((1, 128)),
                  _full((128, 128)), _full((1, 128))],
        out_specs=row(128),
        out_shape=jax.ShapeDtypeStruct((N, 128), jnp.float32),
    )(y, st, p['c_bn_g'].reshape(1, -1), p['c_bn_b'].reshape(1, -1),
      p['c_w2'], p['c_b2'].reshape(1, -1))


def _recon_body(h_ref, w1_ref, b1_ref, w2_ref, b2_ref,
                w3_ref, b3_ref, w4_ref, b4_ref, out_ref):
    t = jax.nn.relu(jnp.dot(h_ref[...], w1_ref[...], preferred_element_type=jnp.float32) + b1_ref[...])
    enc = jnp.dot(t, w2_ref[...], preferred_element_type=jnp.float32) + b2_ref[...]
    t2 = jax.nn.relu(jnp.dot(enc, w3_ref[...], preferred_element_type=jnp.float32) + b3_ref[...])
    out_ref[...] = jnp.dot(t2, w4_ref[...], preferred_element_type=jnp.float32) + b4_ref[...]


def _recon(h, p):
    row = lambda c: pl.BlockSpec((_BN, c), lambda i: (i, 0))
    return pl.pallas_call(
        _recon_body,
        grid=(N // _BN,),
        in_specs=[row(128),
                  _full((128, 128)), _full((1, 128)),
                  _full((128, 64)), _full((1, 64)),
                  _full((64, 128)), _full((1, 128)),
                  _full((128, 128)), _full((1, 128))],
        out_specs=row(128),
        out_shape=jax.ShapeDtypeStruct((N, 128), jnp.float32),
    )(h, p['r_e_w1'], p['r_e_b1'].reshape(1, -1),
      p['r_e_w2'], p['r_e_b2'].reshape(1, -1),
      p['r_d_w1'], p['r_d_b1'].reshape(1, -1),
      p['r_d_w2'], p['r_d_b2'].reshape(1, -1))


_MESH = plsc.VectorSubcoreMesh(core_axis_name="c", subcore_axis_name="s",
                               num_cores=NC, num_subcores=NS)
_C = 80
_RPW = E // NW
_NST = _RPW // _C
N_PAD = 10240
_RPT = N_PAD // NS


_CV = 40
_NSV = _RPW // _CV


def _sc_conv(table, src, dst, e2, zeros):
    @functools.partial(
        pl.kernel, mesh=_MESH,
        out_type=jax.ShapeDtypeStruct((NC, N_PAD, 128), jnp.float32),
        scratch_types=[pltpu.VMEM((_CV,), jnp.int32),
                       pltpu.VMEM((_CV,), jnp.int32),
                       pltpu.VMEM((_CV,), jnp.int32),
                       pltpu.VMEM((_CV,), jnp.int32),
                       pltpu.VMEM((_CV, 128), jnp.float32),
                       pltpu.VMEM((_CV, 128), jnp.float32),
                       pltpu.VMEM((_CV, 128), jnp.float32),
                       pltpu.VMEM((_CV, 128), jnp.float32),
                       pltpu.VMEM_SHARED((N_PAD, 128), jnp.float32),
                       pltpu.SemaphoreType.DMA,
                       pltpu.SemaphoreType.DMA,
                       pltpu.SemaphoreType.DMA],
    )
    def k(tab_hbm, src_hbm, dst_hbm, e2_hbm, zero_hbm, sum_hbm,
          s0, s1, d0, d1, x0, x1, e0, e1, acc, gsem, esem, ssem):
        cid = lax.axis_index("c")
        sid = lax.axis_index("s")
        wid = sid * NC + cid
        pltpu.sync_copy(zero_hbm, acc.at[pl.ds(sid * _RPT, _RPT)])
        plsc.subcore_barrier()
        bufs = ((s0, d0, x0, e0), (s1, d1, x1, e1))

        def start(t, sb, xb, eb):
            base = wid * _RPW + t * _CV
            pltpu.sync_copy(src_hbm.at[pl.ds(base, _CV)], sb)
            pltpu.async_copy(e2_hbm.at[pl.ds(base, _CV)], eb, esem)
            pltpu.async_copy(tab_hbm.at[sb], xb, gsem)

        for b in range(2):
            start(b, bufs[b][0], bufs[b][2], bufs[b][3])

        def process(t, sb, db, xb, eb, drain, prefetch):
            base = wid * _RPW + t * _CV
            pltpu.make_async_copy(tab_hbm.at[sb], xb, gsem).wait()
            pltpu.make_async_copy(e2_hbm.at[pl.ds(base, _CV)], eb, esem).wait()
            if drain:
                @pl.when(t >= 2)
                def _dr():
                    pltpu.make_async_copy(xb, acc.at[db], ssem).wait()
            pltpu.sync_copy(dst_hbm.at[pl.ds(base, _CV)], db)

            def row(r, carry):
                for c in range(8):
                    sl = pl.ds(c * 16, 16)
                    v = jnp.maximum(xb[r, sl] + eb[r, sl], 0.0) + 1e-7
                    xb[r, sl] = jnp.minimum(v, 100.0)
                return carry

            lax.fori_loop(0, _CV, row, 0)
            pltpu.async_copy(xb, acc.at[db], ssem, add=True)
            if prefetch:
                @pl.when(t + 2 < _NSV)
                def _pf():
                    start(t + 2, sb, xb, eb)

        def pair(kk, carry):
            for b in range(2):
                t = 2 * kk + b
                process(t, *bufs[b], drain=True, prefetch=True)
            return carry

        lax.fori_loop(0, _NSV // 2 - 1, pair, 0)
        for t in (_NSV - 2, _NSV - 1):
            process(t, *bufs[t % 2], drain=True, prefetch=False)
        for t in (_NSV - 2, _NSV - 1):
            sb, db, xb, eb = bufs[t % 2]
            pltpu.make_async_copy(xb, acc.at[db], ssem).wait()
        plsc.subcore_barrier()
        pltpu.sync_copy(acc.at[pl.ds(sid * _RPT, _RPT)],
                        sum_hbm.at[cid, pl.ds(sid * _RPT, _RPT)])

    return k(table, src, dst, e2, zeros)


def _counts_body(d_ref, out_ref):
    @pl.when(pl.program_id(0) == 0)
    def _init():
        out_ref[...] = jnp.zeros_like(out_ref)

    d = d_ref[...]
    lo = (d & 127) == jax.lax.broadcasted_iota(jnp.int32, (1, 128), 1)
    hi = (d >> 7) == jax.lax.broadcasted_iota(jnp.int32, (1, N_PAD // 128), 1)
    out_ref[...] += jax.lax.dot_general(
        hi.astype(jnp.float32), lo.astype(jnp.float32),
        (((0,), (0,)), ((), ())), preferred_element_type=jnp.float32)


def _counts(dst):
    return pl.pallas_call(
        _counts_body,
        grid=(E // _BE,),
        in_specs=[pl.BlockSpec((_BE, 1), lambda i: (i, 0))],
        out_specs=pl.BlockSpec((N_PAD // 128, 128), lambda i: (0, 0)),
        out_shape=jax.ShapeDtypeStruct((N_PAD // 128, 128), jnp.float32),
    )(dst)


def _fmax_body(hs_ref, hd_ref, ew_ref, src_ref, b_ref, out_ref, bnd_ref):
    @pl.when(pl.program_id(0) == 0)
    def _init():
        out_ref[...] = jnp.full_like(out_ref, -jnp.inf)
        b = b_ref[...]
        bnd_ref[0] = 0
        for g in range(1, G):
            bnd_ref[g] = jnp.sum((b < g).astype(jnp.int32))
        bnd_ref[G] = N

    w = jnp.concatenate([hs_ref[...], hd_ref[...]], axis=-1) * ew_ref[...]
    s = src_ref[...]
    for g in range(G):
        mask = (s >= bnd_ref[g]) & (s < bnd_ref[g + 1])
        cand = jnp.where(mask, w, -jnp.inf)
        m = jnp.max(cand, axis=0, keepdims=True)
        out_ref[g:g + 1, :] = jnp.maximum(out_ref[g:g + 1, :], m)


def _fusion_max(hs, hd, ew, src, batch_pad):
    row = lambda c: pl.BlockSpec((_BE, c), lambda i: (i, 0))
    return pl.pallas_call(
        _fmax_body,
        grid=(E // _BE,),
        in_specs=[row(128), row(128), row(1), row(1),
                  pl.BlockSpec((N_PAD // 128, 128), lambda i: (0, 0))],
        out_specs=pl.BlockSpec((G, 256), lambda i: (0, 0)),
        out_shape=jax.ShapeDtypeStruct((G, 256), jnp.float32),
        scratch_shapes=[pltpu.SMEM((G + 1,), jnp.int32)],
    )(hs, hd, ew, src, batch_pad)


def _fhead_body(f_ref, g_ref, b_ref, w_ref, wb_ref, out_ref):
    t = _ln(f_ref[...], g_ref[...], b_ref[...])
    out_ref[...] = jax.nn.sigmoid(
        jnp.dot(t, w_ref[...], preferred_element_type=jnp.float32) + wb_ref[...])


def _fusion_head(fmax, p):
    return pl.pallas_call(
        _fhead_body,
        grid=(1,),
        in_specs=[_full((G, 256)),
                  _full((1, 256)), _full((1, 256)),
                  _full((256, TL)), _full((1, TL))],
        out_specs=_full((G, TL)),
        out_shape=jax.ShapeDtypeStruct((G, TL), jnp.float32),
    )(fmax, p['norm_g'].reshape(1, -1), p['norm_b'].reshape(1, -1),
      p['fuse_w'], p['fuse_b'].reshape(1, -1))


def _contr_a_body(h_ref, w_ref, b_ref, y_ref, st_ref):
    @pl.when(pl.program_id(0) == 0)
    def _init():
        st_ref[...] = jnp.zeros_like(st_ref)

    y = jnp.dot(h_ref[...], w_ref[...], preferred_element_type=jnp.float32) + b_ref[...]
    y_ref[...] = y
    st_ref[0:1, :] += jnp.sum(y, axis=0, keepdims=True)
    st_ref[1:2, :] += jnp.sum(y * y, axis=0, keepdims=True)


def _contr_b_body(y_ref, st_ref, g_ref, b_ref, w_ref, wb_ref, out_ref):
    mean = st_ref[0:1, :] / N
    var = st_ref[1:2, :] / N - mean * mean
    xh = (y_ref[...] - mean) / jnp.sqrt(var + 1e-5) * g_ref[...] + b_ref[...]
    xh = jax.nn.relu(xh)
    out_ref[...] = jnp.dot(xh, w_ref[...], preferred_element_type=jnp.float32) + wb_ref[...]


def _contrastive(h, p):
    row = lambda c: pl.BlockSpec((_BN, c), lambda i: (i, 0))
    y, st = pl.pallas_call(
        _contr_a_body,
        grid=(N // _BN,),
        in_specs=[row(128), _full((128, 128)), _full((1, 128))],
        out_specs=[row(128), pl.BlockSpec((8, 128), lambda i: (0, 0))],
        out_shape=[jax.ShapeDtypeStruct((N, 128), jnp.float32),
                   jax.ShapeDtypeStruct((8, 128), jnp.float32)],
    )(h, p['c_w1'], p['c_b1'].reshape(1, -1))
    return pl.pallas_call(
        _contr_b_body,
        grid=(N // _BN,),
        in_specs=[row(128), _full((8, 128)), _full((1, 128)), _full((1, 128)),
                  _full((128, 128)), _full((1, 128))],
        out_specs=row(128),
        out_shape=jax.ShapeDtypeStruct((N, 128), jnp.float32),
    )(y, st, p['c_bn_g'].reshape(1, -1), p['c_bn_b'].reshape(1, -1),
      p['c_w2'], p['c_b2'].reshape(1, -1))


def _recon_body(h_ref, w1_ref, b1_ref, w2_ref, b2_ref,
                w3_ref, b3_ref, w4_ref, b4_ref, out_ref):
    t = jax.nn.relu(jnp.dot(h_ref[...], w1_ref[...], preferred_element_type=jnp.float32) + b1_ref[...])
    enc = jnp.dot(t, w2_ref[...], preferred_element_type=jnp.float32) + b2_ref[...]
    t2 = jax.nn.relu(jnp.dot(enc, w3_ref[...], preferred_element_type=jnp.float32) + b3_ref[...])
    out_ref[...] = jnp.dot(t2, w4_ref[...], preferred_element_type=jnp.float32) + b4_ref[...]


def _recon(h, p):
    row = lambda c: pl.BlockSpec((_BN, c), lambda i: (i, 0))
    return pl.pallas_call(
        _recon_body,
        grid=(N // _BN,),
        in_specs=[row(128),
                  _full((128, 128)), _full((1, 128)),
                  _full((128, 64)), _full((1, 64)),
                  _full((64, 128)), _full((1, 128)),
                  _full((128, 128)), _full((1, 128))],
        out_specs=row(128),
        out_shape=jax.ShapeDtypeStruct((N, 128), jnp.float32),
    )(h, p['r_e_w1'], p['r_e_b1'].reshape(1, -1),
      p['r_e_w2'], p['r_e_b2'].reshape(1, -1),
      p['r_d_w1'], p['r_d_b1'].reshape(1, -1),
      p['r_d_w2'], p['r_d_b2'].reshape(1, -1))


_MESH = plsc.VectorSubcoreMesh(core_axis_name="c", subcore_axis_name="s",
                               num_cores=NC, num_subcores=NS)
_C = 80
_RPW = E // NW
_NST = _RPW // _C
N_PAD = 10240
_RPT = N_PAD // NS


_CV = 80
_NSV = _RPW // _CV


def _sc_conv(table, src, dst, e2, zeros):
    @functools.partial(
        pl.kernel, mesh=_MESH,
        out_type=jax.ShapeDtypeStruct((NC, N_PAD, 128), jnp.float32),
        scratch_types=[pltpu.VMEM((_CV,), jnp.int32),
                       pltpu.VMEM((_CV,), jnp.int32),
                       pltpu.VMEM((_CV,), jnp.int32),
                       pltpu.VMEM((_CV,), jnp.int32),
                       pltpu.VMEM((_CV, 128), jnp.float32),
                       pltpu.VMEM((_CV, 128), jnp.float32),
                       pltpu.VMEM((_CV, 128), jnp.float32),
                       pltpu.VMEM_SHARED((N_PAD, 128), jnp.float32),
                       pltpu.SemaphoreType.DMA,
                       pltpu.SemaphoreType.DMA,
                       pltpu.SemaphoreType.DMA],
    )
    def k(tab_hbm, src_hbm, dst_hbm, e2_hbm, zero_hbm, sum_hbm,
          s0, s1, d0, d1, x0, x1, eb, acc, gsem, esem, ssem):
        cid = lax.axis_index("c")
        sid = lax.axis_index("s")
        wid = sid * NC + cid
        pltpu.sync_copy(zero_hbm, acc.at[pl.ds(sid * _RPT, _RPT)])
        plsc.subcore_barrier()
        bufs = ((s0, d0, x0), (s1, d1, x1))

        def e2_load(t):
            base = wid * _RPW + t * _CV
            pltpu.async_copy(e2_hbm.at[pl.ds(base, _CV)], eb, esem)

        def start(t, sb, xb):
            base = wid * _RPW + t * _CV
            pltpu.sync_copy(src_hbm.at[pl.ds(base, _CV)], sb)
            pltpu.async_copy(tab_hbm.at[sb], xb, gsem)

        e2_load(0)
        for b in range(2):
            start(b, bufs[b][0], bufs[b][2])

        def process(t, sb, db, xb, drain, prefetch):
            base = wid * _RPW + t * _CV
            pltpu.make_async_copy(tab_hbm.at[sb], xb, gsem).wait()
            pltpu.make_async_copy(e2_hbm.at[pl.ds(base, _CV)], eb, esem).wait()
            if drain:
                @pl.when(t >= 2)
                def _dr():
                    pltpu.make_async_copy(xb, acc.at[db], ssem).wait()
            pltpu.sync_copy(dst_hbm.at[pl.ds(base, _CV)], db)

            def row(r, carry):
                for c in range(8):
                    sl = pl.ds(c * 16, 16)
                    v = jnp.maximum(xb[r, sl] + eb[r, sl], 0.0) + 1e-7
                    xb[r, sl] = jnp.minimum(v, 100.0)
                return carry

            lax.fori_loop(0, _CV, row, 0)

            @pl.when(t + 1 < _NSV)
            def _el():
                e2_load(t + 1)

            pltpu.async_copy(xb, acc.at[db], ssem, add=True)
            if prefetch:
                @pl.when(t + 2 < _NSV)
                def _pf():
                    start(t + 2, sb, xb)

        def pair(kk, carry):
            for b in range(2):
                t = 2 * kk + b
                process(t, *bufs[b], drain=True, prefetch=True)
            return carry

        lax.fori_loop(0, (_NSV - 1) // 2, pair, 0)
        tl = _NSV - 1
        process(tl, *bufs[tl % 2], drain=True, prefetch=False)
        for t in (_NSV - 2, _NSV - 1):
            sb, db, xb = bufs[t % 2]
            pltpu.make_async_copy(xb, acc.at[db], ssem).wait()
        plsc.subcore_barrier()
        pltpu.sync_copy(acc.at[pl.ds(sid * _RPT, _RPT)],
                        sum_hbm.at[cid, pl.ds(sid * _RPT, _RPT)])

    return k(table, src, dst, e2, zeros)


_CF = 80
_NSF = _RPW // _CF


def _sc_gather_fusion(h, src, dst):
    @functools.partial(
        pl.kernel, mesh=_MESH,
        out_type=[jax.ShapeDtypeStruct((E, 128), jnp.float32),
                  jax.ShapeDtypeStruct((E, 128), jnp.float32)],
        scratch_types=[pltpu.VMEM((_CF,), jnp.int32),
                       pltpu.VMEM((_CF,), jnp.int32),
                       pltpu.VMEM((_CF,), jnp.int32),
                       pltpu.VMEM((_CF,), jnp.int32),
                       pltpu.VMEM((_CF, 128), jnp.float32),
                       pltpu.VMEM((_CF, 128), jnp.float32),
                       pltpu.VMEM((_CF, 128), jnp.float32),
                       pltpu.VMEM((_CF, 128), jnp.float32),
                       pltpu.SemaphoreType.DMA],
    )
    def k(h_hbm, src_hbm, dst_hbm, hs_hbm, hd_hbm,
          s0, s1, d0, d1, hs0, hs1, hd0, hd1, sem):
        wid = lax.axis_index("s") * NC + lax.axis_index("c")
        bufs = ((s0, d0, hs0, hd0), (s1, d1, hs1, hd1))

        def start(t, sb, db, hsb, hdb):
            base = wid * _RPW + t * _CF
            pltpu.sync_copy(src_hbm.at[pl.ds(base, _CF)], sb)
            pltpu.sync_copy(dst_hbm.at[pl.ds(base, _CF)], db)
            pltpu.async_copy(h_hbm.at[sb], hsb, sem)
            pltpu.async_copy(h_hbm.at[db], hdb, sem)

        for b in range(2):
            start(b, *bufs[b])

        def finish(t, sb, db, hsb, hdb):
            pltpu.make_async_copy(h_hbm.at[sb], hsb, sem).wait()
            pltpu.make_async_copy(h_hbm.at[db], hdb, sem).wait()
            base = wid * _RPW + t * _CF
            pltpu.sync_copy(hsb, hs_hbm.at[pl.ds(base, _CF)])
            pltpu.sync_copy(hdb, hd_hbm.at[pl.ds(base, _CF)])

        def pair(kk, carry):
            for b in range(2):
                t = 2 * kk + b
                finish(t, *bufs[b])

                @pl.when(t + 2 < _NSF)
                def _pf():
                    start(t + 2, *bufs[b])
            return carry

        lax.fori_loop(0, (_NSF - 1) // 2, pair, 0)
        tl = _NSF - 1
        finish(tl, *bufs[tl % 2])

    return k(h, src, dst)


def kernel(x, edge_index, time_attr, aux_info, pos, batch, params):
    p = params
    src, dst = edge_index[0], edge_index[1]
    e2_1, e2_2, ew = _edge_pre(time_attr, aux_info, p)
    zeros = jnp.zeros((_RPT, 128), jnp.float32)
    batch_pad = jnp.pad(batch, (0, N_PAD - N), constant_values=G).reshape(N_PAD // 128, 128)
    cnt = _counts(dst.reshape(E, 1)).reshape(N_PAD, 1)[:N]

    sums1 = _sc_conv(x, src, dst, e2_1, zeros)
    h1 = _node_update(x, sums1[0, :N], sums1[1, :N], cnt, p['conv1'])

    sums2 = _sc_conv(h1, src, dst, e2_2, zeros)
    h = _node_update(h1, sums2[0, :N], sums2[1, :N], cnt, p['conv2'])

    hs, hd = _sc_gather_fusion(h, src, dst)
    fmax = _fusion_max(hs, hd, ew, src.reshape(E, 1), batch_pad)
    fusion = _fusion_head(fmax, p)

    contrastive = _contrastive(h, p)
    recon = _recon(h, p)
    return (h, fusion, contrastive, recon)
